# Initial kernel scaffold; baseline (speedup 1.0000x reference)
import jax
import jax.numpy as jnp
from jax.experimental import pallas as pl


def kernel(x, edge_index, W1, b1, W2, b2, W3, b3, eW1, eb1, g1, be1, eW2, eb2, g2, be2, eW3, eb3):
    raise NotImplementedError("write your pallas kernel here")


# speedup vs baseline: 6.8071x; 6.8071x over previous
import functools

import jax
import jax.numpy as jnp
from jax import lax
from jax.experimental import pallas as pl
from jax.experimental.pallas import tpu as pltpu
from jax.experimental.pallas import tpu_sc as plsc

NC = 2
NS = 16
NW = NC * NS

C = 80


def _mesh():
    return plsc.VectorSubcoreMesh(core_axis_name="c", subcore_axis_name="s")


def _deg_partials(ones, dst3, zeros16, n, ch):
    rows = n // NS

    @functools.partial(
        pl.kernel,
        out_type=jax.ShapeDtypeStruct((NC, n, 16), jnp.float32),
        mesh=_mesh(),
        compiler_params=pltpu.CompilerParams(use_tc_tiling_on_sc=False),
        scratch_types=[
            pltpu.VMEM((ch, C), jnp.int32),
            pltpu.VMEM((C, 16), jnp.float32),
            pltpu.VMEM_SHARED((n, 16), jnp.float32),
        ],
    )
    def k(ones_hbm, dst_hbm, zero_hbm, out_hbm, dst_v, ones_v, deg_sh):
        c = lax.axis_index("c")
        s = lax.axis_index("s")
        wid = s * NC + c
        pltpu.sync_copy(zero_hbm.at[pl.ds(s * rows, rows)],
                        deg_sh.at[pl.ds(s * rows, rows)])
        pltpu.sync_copy(dst_hbm.at[wid], dst_v)
        pltpu.sync_copy(ones_hbm, ones_v)
        plsc.subcore_barrier()

        def body(j, carry):
            pltpu.sync_copy(ones_v, deg_sh.at[dst_v.at[j]], add=True)
            return carry

        lax.fori_loop(0, ch, body, 0)
        plsc.subcore_barrier()
        pltpu.sync_copy(deg_sh.at[pl.ds(s * rows, rows)],
                        out_hbm.at[c, pl.ds(s * rows, rows)])

    return k(ones, dst3, zeros16)


def _seg_sum(u, src3, dst3, zeros64, n, h, ch):
    rows = n // NS

    @functools.partial(
        pl.kernel,
        out_type=jax.ShapeDtypeStruct((NC, n, h), jnp.float32),
        mesh=_mesh(),
        compiler_params=pltpu.CompilerParams(use_tc_tiling_on_sc=False),
        scratch_types=[
            pltpu.VMEM((ch, C), jnp.int32),
            pltpu.VMEM((ch, C), jnp.int32),
            pltpu.VMEM((C, h), jnp.float32),
            pltpu.VMEM_SHARED((n, h), jnp.float32),
        ],
    )
    def k(u_hbm, src_hbm, dst_hbm, zero_hbm, out_hbm, src_v, dst_v, gath_v,
          acc_sh):
        c = lax.axis_index("c")
        s = lax.axis_index("s")
        wid = s * NC + c
        pltpu.sync_copy(zero_hbm.at[pl.ds(s * rows, rows)],
                        acc_sh.at[pl.ds(s * rows, rows)])
        pltpu.sync_copy(src_hbm.at[wid], src_v)
        pltpu.sync_copy(dst_hbm.at[wid], dst_v)
        plsc.subcore_barrier()

        def body(j, carry):
            pltpu.sync_copy(u_hbm.at[src_v.at[j]], gath_v)
            pltpu.sync_copy(gath_v, acc_sh.at[dst_v.at[j]], add=True)
            return carry

        lax.fori_loop(0, ch, body, 0)
        plsc.subcore_barrier()
        pltpu.sync_copy(acc_sh.at[pl.ds(s * rows, rows)],
                        out_hbm.at[c, pl.ds(s * rows, rows)])

    return k(u, src3, dst3, zeros64)


def _edge_head(p, q, src3, dst3, n, h, e, ch):
    ew = e // NW

    @functools.partial(
        pl.kernel,
        out_type=(
            jax.ShapeDtypeStruct((e, h), jnp.float32),
            jax.ShapeDtypeStruct((NW, 8, 16), jnp.float32),
        ),
        mesh=_mesh(),
        compiler_params=pltpu.CompilerParams(use_tc_tiling_on_sc=False),
        scratch_types=[
            pltpu.VMEM((ch, C), jnp.int32),
            pltpu.VMEM((ch, C), jnp.int32),
            pltpu.VMEM((C, h), jnp.float32),
            pltpu.VMEM((C, h), jnp.float32),
            pltpu.VMEM((8, 16), jnp.float32),
        ],
    )
    def k(p_hbm, q_hbm, src_hbm, dst_hbm, z1_hbm, st_hbm, src_v, dst_v, a_v,
          b_v, st_v):
        c = lax.axis_index("c")
        s = lax.axis_index("s")
        wid = s * NC + c
        pltpu.sync_copy(src_hbm.at[wid], src_v)
        pltpu.sync_copy(dst_hbm.at[wid], dst_v)
        zero = jnp.zeros((16,), jnp.float32)

        def chunk(j, carry):
            pltpu.sync_copy(p_hbm.at[src_v.at[j]], a_v)
            pltpu.sync_copy(q_hbm.at[dst_v.at[j]], b_v)

            def row(r, cr):
                acc = list(cr)
                for kk in range(4):
                    sl = pl.ds(kk * 16, 16)
                    z = jnp.maximum(a_v[r, sl] + b_v[r, sl], 0.0)
                    a_v[r, sl] = z
                    acc[kk] = acc[kk] + z
                    acc[4 + kk] = acc[4 + kk] + z * z
                return tuple(acc)

            cr = lax.fori_loop(0, C, row, carry)
            pltpu.sync_copy(a_v, z1_hbm.at[pl.ds(wid * ew + j * C, C)])
            return cr

        carry = lax.fori_loop(0, ch, chunk, (zero,) * 8)
        for kk in range(8):
            st_v[kk, :] = carry[kk]
        pltpu.sync_copy(st_v, st_hbm.at[wid])

    return k(p, q, src3, dst3)


def _tc_in(x, w1, degp, n, d, h):
    bn = 640

    def body(x_b, w_r, deg_b, g_b, u_b, di_b):
        deg = deg_b[...]
        dtot = deg[0, :, 0:1] + deg[1, :, 0:1] + 1.0
        di = lax.rsqrt(dtot)
        g = jnp.dot(x_b[...], w_r[...], preferred_element_type=jnp.float32)
        g_b[...] = g
        u_b[...] = g * di
        di_b[...] = jnp.broadcast_to(di, di_b.shape)

    return pl.pallas_call(
        body,
        grid=(n // bn,),
        in_specs=[
            pl.BlockSpec((bn, d), lambda i: (i, 0)),
            pl.BlockSpec((d, h), lambda i: (0, 0)),
            pl.BlockSpec((NC, bn, 16), lambda i: (0, i, 0)),
        ],
        out_specs=[
            pl.BlockSpec((bn, h), lambda i: (i, 0)),
            pl.BlockSpec((bn, h), lambda i: (i, 0)),
            pl.BlockSpec((bn, 16), lambda i: (i, 0)),
        ],
        out_shape=[
            jax.ShapeDtypeStruct((n, h), jnp.float32),
            jax.ShapeDtypeStruct((n, h), jnp.float32),
            jax.ShapeDtypeStruct((n, 16), jnp.float32),
        ],
    )(x, w1, degp)


def _tc_conv(acc, g, di, b, w, n, h):
    bn = 640

    def body(acc_b, g_b, di_b, b_r, w_r, gn_b, un_b):
        a = acc_b[...]
        dv = di_b[...][:, 0:1]
        hh = jnp.maximum(dv * (a[0] + a[1]) + dv * dv * g_b[...] + b_r[...],
                         0.0)
        gn = jnp.dot(hh, w_r[...], preferred_element_type=jnp.float32)
        gn_b[...] = gn
        un_b[...] = gn * dv

    return pl.pallas_call(
        body,
        grid=(n // bn,),
        in_specs=[
            pl.BlockSpec((NC, bn, h), lambda i: (0, i, 0)),
            pl.BlockSpec((bn, h), lambda i: (i, 0)),
            pl.BlockSpec((bn, 16), lambda i: (i, 0)),
            pl.BlockSpec((1, h), lambda i: (0, 0)),
            pl.BlockSpec((h, h), lambda i: (0, 0)),
        ],
        out_specs=[
            pl.BlockSpec((bn, h), lambda i: (i, 0)),
            pl.BlockSpec((bn, h), lambda i: (i, 0)),
        ],
        out_shape=[
            jax.ShapeDtypeStruct((n, h), jnp.float32),
            jax.ShapeDtypeStruct((n, h), jnp.float32),
        ],
    )(acc, g, di, b, w)


def _tc_pq(acc, g, di, b, ea, eb, eb1, n, h):
    bn = 640

    def body(acc_b, g_b, di_b, b_r, ea_r, eb_r, eb1_r, p_b, q_b):
        a = acc_b[...]
        dv = di_b[...][:, 0:1]
        h3 = dv * (a[0] + a[1]) + dv * dv * g_b[...] + b_r[...]
        p_b[...] = (jnp.dot(h3, ea_r[...], preferred_element_type=jnp.float32)
                    + eb1_r[...])
        q_b[...] = jnp.dot(h3, eb_r[...], preferred_element_type=jnp.float32)

    return pl.pallas_call(
        body,
        grid=(n // bn,),
        in_specs=[
            pl.BlockSpec((NC, bn, h), lambda i: (0, i, 0)),
            pl.BlockSpec((bn, h), lambda i: (i, 0)),
            pl.BlockSpec((bn, 16), lambda i: (i, 0)),
            pl.BlockSpec((1, h), lambda i: (0, 0)),
            pl.BlockSpec((h, h), lambda i: (0, 0)),
            pl.BlockSpec((h, h), lambda i: (0, 0)),
            pl.BlockSpec((1, h), lambda i: (0, 0)),
        ],
        out_specs=[
            pl.BlockSpec((bn, h), lambda i: (i, 0)),
            pl.BlockSpec((bn, h), lambda i: (i, 0)),
        ],
        out_shape=[
            jax.ShapeDtypeStruct((n, h), jnp.float32),
            jax.ShapeDtypeStruct((n, h), jnp.float32),
        ],
    )(acc, g, di, b, ea, eb, eb1)


def _tc_mlp_mid(z1, s, t, w, b, e, hin, hout):
    be = 512

    def body(z_b, s_r, t_r, w_r, b_r, z2_b, st_b):
        zn = z_b[...] * s_r[...] + t_r[...]
        z2 = jnp.maximum(
            jnp.dot(zn, w_r[...], preferred_element_type=jnp.float32)
            + b_r[...], 0.0)
        z2_b[...] = z2
        st = jnp.concatenate(
            [jnp.sum(z2, axis=0, keepdims=True),
             jnp.sum(z2 * z2, axis=0, keepdims=True)], axis=0)

        @pl.when(pl.program_id(0) == 0)
        def _():
            st_b[...] = st

        @pl.when(pl.program_id(0) != 0)
        def _():
            st_b[...] = st_b[...] + st

    return pl.pallas_call(
        body,
        grid=(e // be,),
        in_specs=[
            pl.BlockSpec((be, hin), lambda i: (i, 0)),
            pl.BlockSpec((1, hin), lambda i: (0, 0)),
            pl.BlockSpec((1, hin), lambda i: (0, 0)),
            pl.BlockSpec((hin, hout), lambda i: (0, 0)),
            pl.BlockSpec((1, hout), lambda i: (0, 0)),
        ],
        out_specs=[
            pl.BlockSpec((be, hout), lambda i: (i, 0)),
            pl.BlockSpec((2, hout), lambda i: (0, 0)),
        ],
        out_shape=[
            jax.ShapeDtypeStruct((e, hout), jnp.float32),
            jax.ShapeDtypeStruct((2, hout), jnp.float32),
        ],
    )(z1, s, t, w, b)


def _tc_mlp_out(z2, s, t, w, b, e, hin, hout):
    be = 512

    def body(z_b, s_r, t_r, w_r, b_r, o_b):
        zn = z_b[...] * s_r[...] + t_r[...]
        o_b[...] = (jnp.dot(zn, w_r[...],
                            preferred_element_type=jnp.float32) + b_r[...])

    return pl.pallas_call(
        body,
        grid=(e // be,),
        in_specs=[
            pl.BlockSpec((be, hin), lambda i: (i, 0)),
            pl.BlockSpec((1, hin), lambda i: (0, 0)),
            pl.BlockSpec((1, hin), lambda i: (0, 0)),
            pl.BlockSpec((hin, hout), lambda i: (0, 0)),
            pl.BlockSpec((1, hout), lambda i: (0, 0)),
        ],
        out_specs=pl.BlockSpec((be, hout), lambda i: (i, 0)),
        out_shape=jax.ShapeDtypeStruct((e, hout), jnp.float32),
    )(z2, s, t, w, b)


def kernel(x, edge_index, W1, b1, W2, b2, W3, b3, eW1, eb1, g1, be1, eW2,
           eb2, g2, be2, eW3, eb3):
    n, d = x.shape
    e = edge_index.shape[1]
    h = W1.shape[1]
    ew = e // NW
    ch = ew // C
    np_ = ((n + 639) // 640) * 640

    src3 = edge_index[0].reshape(NW, ch, C)
    dst3 = edge_index[1].reshape(NW, ch, C)
    zeros64 = jnp.zeros((np_, h), jnp.float32)
    zeros16 = jnp.zeros((np_, 16), jnp.float32)
    ones = jnp.ones((C, 16), jnp.float32)
    xp = jnp.pad(x, ((0, np_ - n), (0, 0)))

    degp = _deg_partials(ones, dst3, zeros16, np_, ch)
    g_1, u_1, di = _tc_in(xp, W1, degp, np_, d, h)
    acc1 = _seg_sum(u_1, src3, dst3, zeros64, np_, h, ch)
    g_2, u_2 = _tc_conv(acc1, g_1, di, b1.reshape(1, h), W2, np_, h)
    acc2 = _seg_sum(u_2, src3, dst3, zeros64, np_, h, ch)
    g_3, u_3 = _tc_conv(acc2, g_2, di, b2.reshape(1, h), W3, np_, h)
    acc3 = _seg_sum(u_3, src3, dst3, zeros64, np_, h, ch)
    P, Q = _tc_pq(acc3, g_3, di, b3.reshape(1, h), eW1[:h], eW1[h:],
                  eb1.reshape(1, h), np_, h)

    z1, st1 = _edge_head(P, Q, src3, dst3, np_, h, e, ch)

    ssum = st1[:, :4, :].reshape(NW, h).sum(0)
    ssq = st1[:, 4:, :].reshape(NW, h).sum(0)
    mean1 = ssum / e
    var1 = ssq / e - mean1 * mean1
    s1 = g1 / jnp.sqrt(var1 + 1e-5)
    t1 = be1 - mean1 * s1

    z2, st2 = _tc_mlp_mid(z1, s1.reshape(1, -1), t1.reshape(1, -1), eW2,
                          eb2.reshape(1, -1), e, h, eW2.shape[1])

    mean2 = st2[0] / e
    var2 = st2[1] / e - mean2 * mean2
    s2 = g2 / jnp.sqrt(var2 + 1e-5)
    t2 = be2 - mean2 * s2

    return _tc_mlp_out(z2, s2.reshape(1, -1), t2.reshape(1, -1), eW3,
                       eb3.reshape(1, -1), e, eW2.shape[1], eW3.shape[1])


# speedup vs baseline: 11.0047x; 1.6167x over previous
import functools

import jax
import jax.numpy as jnp
from jax import lax
from jax.experimental import pallas as pl
from jax.experimental.pallas import tpu as pltpu
from jax.experimental.pallas import tpu_sc as plsc

NC = 2
NS = 16
NW = NC * NS

C = 80


def _mesh():
    return plsc.VectorSubcoreMesh(core_axis_name="c", subcore_axis_name="s")


def _deg_partials(ones, dst3, zeros16, n, ch):
    rows = n // NS

    @functools.partial(
        pl.kernel,
        out_type=jax.ShapeDtypeStruct((NC, n, 16), jnp.float32),
        mesh=_mesh(),
        compiler_params=pltpu.CompilerParams(use_tc_tiling_on_sc=False),
        scratch_types=[
            pltpu.VMEM((ch, C), jnp.int32),
            pltpu.VMEM((C, 16), jnp.float32),
            pltpu.VMEM_SHARED((n, 16), jnp.float32),
        ],
    )
    def k(ones_hbm, dst_hbm, zero_hbm, out_hbm, dst_v, ones_v, deg_sh):
        c = lax.axis_index("c")
        s = lax.axis_index("s")
        wid = s * NC + c
        pltpu.sync_copy(zero_hbm.at[pl.ds(s * rows, rows)],
                        deg_sh.at[pl.ds(s * rows, rows)])
        pltpu.sync_copy(dst_hbm.at[wid], dst_v)
        pltpu.sync_copy(ones_hbm, ones_v)
        plsc.subcore_barrier()

        def body(j, carry):
            pltpu.sync_copy(ones_v, deg_sh.at[dst_v.at[j]], add=True)
            return carry

        lax.fori_loop(0, ch, body, 0)
        plsc.subcore_barrier()
        pltpu.sync_copy(deg_sh.at[pl.ds(s * rows, rows)],
                        out_hbm.at[c, pl.ds(s * rows, rows)])

    return k(ones, dst3, zeros16)


def _seg_sum(u, src3, dst3, zeros64, n, h, ch):
    rows = n // NS

    @functools.partial(
        pl.kernel,
        out_type=jax.ShapeDtypeStruct((NC, n, h), jnp.float32),
        mesh=_mesh(),
        compiler_params=pltpu.CompilerParams(use_tc_tiling_on_sc=False),
        scratch_types=[
            pltpu.VMEM((ch, C), jnp.int32),
            pltpu.VMEM((ch, C), jnp.int32),
            pltpu.VMEM((C, h), jnp.float32),
            pltpu.VMEM_SHARED((n, h), jnp.float32),
        ],
    )
    def k(u_hbm, src_hbm, dst_hbm, zero_hbm, out_hbm, src_v, dst_v, gath_v,
          acc_sh):
        c = lax.axis_index("c")
        s = lax.axis_index("s")
        wid = s * NC + c
        pltpu.sync_copy(zero_hbm.at[pl.ds(s * rows, rows)],
                        acc_sh.at[pl.ds(s * rows, rows)])
        pltpu.sync_copy(src_hbm.at[wid], src_v)
        pltpu.sync_copy(dst_hbm.at[wid], dst_v)
        plsc.subcore_barrier()

        def body(j, carry):
            pltpu.sync_copy(u_hbm.at[src_v.at[j]], gath_v)
            pltpu.sync_copy(gath_v, acc_sh.at[dst_v.at[j]], add=True)
            return carry

        lax.fori_loop(0, ch, body, 0)
        plsc.subcore_barrier()
        pltpu.sync_copy(acc_sh.at[pl.ds(s * rows, rows)],
                        out_hbm.at[c, pl.ds(s * rows, rows)])

    return k(u, src3, dst3, zeros64)


def _edge_head(p, q, src3, dst3, n, h, e, ch):
    ew = e // NW

    @functools.partial(
        pl.kernel,
        out_type=(
            jax.ShapeDtypeStruct((e, h), jnp.float32),
            jax.ShapeDtypeStruct((NW, 8, 16), jnp.float32),
        ),
        mesh=_mesh(),
        compiler_params=pltpu.CompilerParams(use_tc_tiling_on_sc=False),
        scratch_types=[
            pltpu.VMEM((ch, C), jnp.int32),
            pltpu.VMEM((ch, C), jnp.int32),
            pltpu.VMEM((C, h), jnp.float32),
            pltpu.VMEM((C, h), jnp.float32),
            pltpu.VMEM((8, 16), jnp.float32),
        ],
    )
    def k(p_hbm, q_hbm, src_hbm, dst_hbm, z1_hbm, st_hbm, src_v, dst_v, a_v,
          b_v, st_v):
        c = lax.axis_index("c")
        s = lax.axis_index("s")
        wid = s * NC + c
        pltpu.sync_copy(src_hbm.at[wid], src_v)
        pltpu.sync_copy(dst_hbm.at[wid], dst_v)
        zero = jnp.zeros((16,), jnp.float32)

        def chunk(j, carry):
            pltpu.sync_copy(p_hbm.at[src_v.at[j]], a_v)
            pltpu.sync_copy(q_hbm.at[dst_v.at[j]], b_v)

            def row(r, cr):
                acc = list(cr)
                for kk in range(4):
                    sl = pl.ds(kk * 16, 16)
                    z = jnp.maximum(a_v[r, sl] + b_v[r, sl], 0.0)
                    a_v[r, sl] = z
                    acc[kk] = acc[kk] + z
                    acc[4 + kk] = acc[4 + kk] + z * z
                return tuple(acc)

            cr = lax.fori_loop(0, C, row, carry)
            pltpu.sync_copy(a_v, z1_hbm.at[pl.ds(wid * ew + j * C, C)])
            return cr

        carry = lax.fori_loop(0, ch, chunk, (zero,) * 8)
        for kk in range(8):
            st_v[kk, :] = carry[kk]
        pltpu.sync_copy(st_v, st_hbm.at[wid])

    return k(p, q, src3, dst3)


def _tc_in(x, w1, degp, n, d, h):
    bn = 2560

    def body(x_b, w_r, deg_b, g_b, u_b, di_b):
        deg = deg_b[...]
        dtot = deg[0, :, 0:1] + deg[1, :, 0:1] + 1.0
        di = lax.rsqrt(dtot)
        g = jnp.dot(x_b[...], w_r[...], preferred_element_type=jnp.float32)
        g_b[...] = g
        u_b[...] = g * di
        di_b[...] = jnp.broadcast_to(di, di_b.shape)

    return pl.pallas_call(
        body,
        grid=(n // bn,),
        in_specs=[
            pl.BlockSpec((bn, d), lambda i: (i, 0)),
            pl.BlockSpec((d, h), lambda i: (0, 0)),
            pl.BlockSpec((NC, bn, 16), lambda i: (0, i, 0)),
        ],
        out_specs=[
            pl.BlockSpec((bn, h), lambda i: (i, 0)),
            pl.BlockSpec((bn, h), lambda i: (i, 0)),
            pl.BlockSpec((bn, 16), lambda i: (i, 0)),
        ],
        out_shape=[
            jax.ShapeDtypeStruct((n, h), jnp.float32),
            jax.ShapeDtypeStruct((n, h), jnp.float32),
            jax.ShapeDtypeStruct((n, 16), jnp.float32),
        ],
    )(x, w1, degp)


def _tc_conv(acc, g, di, b, w, n, h):
    bn = 2560

    def body(acc_b, g_b, di_b, b_r, w_r, gn_b, un_b):
        a = acc_b[...]
        dv = di_b[...][:, 0:1]
        hh = jnp.maximum(dv * (a[0] + a[1]) + dv * dv * g_b[...] + b_r[...],
                         0.0)
        gn = jnp.dot(hh, w_r[...], preferred_element_type=jnp.float32)
        gn_b[...] = gn
        un_b[...] = gn * dv

    return pl.pallas_call(
        body,
        grid=(n // bn,),
        in_specs=[
            pl.BlockSpec((NC, bn, h), lambda i: (0, i, 0)),
            pl.BlockSpec((bn, h), lambda i: (i, 0)),
            pl.BlockSpec((bn, 16), lambda i: (i, 0)),
            pl.BlockSpec((1, h), lambda i: (0, 0)),
            pl.BlockSpec((h, h), lambda i: (0, 0)),
        ],
        out_specs=[
            pl.BlockSpec((bn, h), lambda i: (i, 0)),
            pl.BlockSpec((bn, h), lambda i: (i, 0)),
        ],
        out_shape=[
            jax.ShapeDtypeStruct((n, h), jnp.float32),
            jax.ShapeDtypeStruct((n, h), jnp.float32),
        ],
    )(acc, g, di, b, w)


def _tc_pq(acc, g, di, b, ea, eb, eb1, n, h):
    bn = 2560

    def body(acc_b, g_b, di_b, b_r, ea_r, eb_r, eb1_r, p_b, q_b):
        a = acc_b[...]
        dv = di_b[...][:, 0:1]
        h3 = dv * (a[0] + a[1]) + dv * dv * g_b[...] + b_r[...]
        p_b[...] = (jnp.dot(h3, ea_r[...], preferred_element_type=jnp.float32)
                    + eb1_r[...])
        q_b[...] = jnp.dot(h3, eb_r[...], preferred_element_type=jnp.float32)

    return pl.pallas_call(
        body,
        grid=(n // bn,),
        in_specs=[
            pl.BlockSpec((NC, bn, h), lambda i: (0, i, 0)),
            pl.BlockSpec((bn, h), lambda i: (i, 0)),
            pl.BlockSpec((bn, 16), lambda i: (i, 0)),
            pl.BlockSpec((1, h), lambda i: (0, 0)),
            pl.BlockSpec((h, h), lambda i: (0, 0)),
            pl.BlockSpec((h, h), lambda i: (0, 0)),
            pl.BlockSpec((1, h), lambda i: (0, 0)),
        ],
        out_specs=[
            pl.BlockSpec((bn, h), lambda i: (i, 0)),
            pl.BlockSpec((bn, h), lambda i: (i, 0)),
        ],
        out_shape=[
            jax.ShapeDtypeStruct((n, h), jnp.float32),
            jax.ShapeDtypeStruct((n, h), jnp.float32),
        ],
    )(acc, g, di, b, ea, eb, eb1)


def _tc_mlp_mid(z1, s, t, w, b, e, hin, hout):
    be = 6400

    def body(z_b, s_r, t_r, w_r, b_r, z2_b, st_b):
        zn = z_b[...] * s_r[...] + t_r[...]
        z2 = jnp.maximum(
            jnp.dot(zn, w_r[...], preferred_element_type=jnp.float32)
            + b_r[...], 0.0)
        z2_b[...] = z2
        st = jnp.concatenate(
            [jnp.sum(z2, axis=0, keepdims=True),
             jnp.sum(z2 * z2, axis=0, keepdims=True)], axis=0)

        @pl.when(pl.program_id(0) == 0)
        def _():
            st_b[...] = st

        @pl.when(pl.program_id(0) != 0)
        def _():
            st_b[...] = st_b[...] + st

    return pl.pallas_call(
        body,
        grid=(e // be,),
        in_specs=[
            pl.BlockSpec((be, hin), lambda i: (i, 0)),
            pl.BlockSpec((1, hin), lambda i: (0, 0)),
            pl.BlockSpec((1, hin), lambda i: (0, 0)),
            pl.BlockSpec((hin, hout), lambda i: (0, 0)),
            pl.BlockSpec((1, hout), lambda i: (0, 0)),
        ],
        out_specs=[
            pl.BlockSpec((be, hout), lambda i: (i, 0)),
            pl.BlockSpec((2, hout), lambda i: (0, 0)),
        ],
        out_shape=[
            jax.ShapeDtypeStruct((e, hout), jnp.float32),
            jax.ShapeDtypeStruct((2, hout), jnp.float32),
        ],
    )(z1, s, t, w, b)


def _tc_mlp_out(z2, s, t, w, b, e, hin, hout):
    be = 6400

    def body(z_b, s_r, t_r, w_r, b_r, o_b):
        zn = z_b[...] * s_r[...] + t_r[...]
        o = (jnp.dot(zn, w_r[...],
                     preferred_element_type=jnp.float32) + b_r[...])
        o_b[...] = o.T

    return pl.pallas_call(
        body,
        grid=(e // be,),
        in_specs=[
            pl.BlockSpec((be, hin), lambda i: (i, 0)),
            pl.BlockSpec((1, hin), lambda i: (0, 0)),
            pl.BlockSpec((1, hin), lambda i: (0, 0)),
            pl.BlockSpec((hin, hout), lambda i: (0, 0)),
            pl.BlockSpec((1, hout), lambda i: (0, 0)),
        ],
        out_specs=pl.BlockSpec((hout, be), lambda i: (0, i)),
        out_shape=jax.ShapeDtypeStruct((hout, e), jnp.float32),
    )(z2, s, t, w, b)


def kernel(x, edge_index, W1, b1, W2, b2, W3, b3, eW1, eb1, g1, be1, eW2,
           eb2, g2, be2, eW3, eb3):
    n, d = x.shape
    e = edge_index.shape[1]
    h = W1.shape[1]
    ew = e // NW
    ch = ew // C
    np_ = ((n + 2559) // 2560) * 2560

    src3 = edge_index[0].reshape(NW, ch, C)
    dst3 = edge_index[1].reshape(NW, ch, C)
    zeros64 = jnp.zeros((np_, h), jnp.float32)
    zeros16 = jnp.zeros((np_, 16), jnp.float32)
    ones = jnp.ones((C, 16), jnp.float32)
    xp = jnp.pad(x, ((0, np_ - n), (0, 0)))

    degp = _deg_partials(ones, dst3, zeros16, np_, ch)
    g_1, u_1, di = _tc_in(xp, W1, degp, np_, d, h)
    acc1 = _seg_sum(u_1, src3, dst3, zeros64, np_, h, ch)
    g_2, u_2 = _tc_conv(acc1, g_1, di, b1.reshape(1, h), W2, np_, h)
    acc2 = _seg_sum(u_2, src3, dst3, zeros64, np_, h, ch)
    g_3, u_3 = _tc_conv(acc2, g_2, di, b2.reshape(1, h), W3, np_, h)
    acc3 = _seg_sum(u_3, src3, dst3, zeros64, np_, h, ch)
    P, Q = _tc_pq(acc3, g_3, di, b3.reshape(1, h), eW1[:h], eW1[h:],
                  eb1.reshape(1, h), np_, h)

    z1, st1 = _edge_head(P, Q, src3, dst3, np_, h, e, ch)

    ssum = st1[:, :4, :].reshape(NW, h).sum(0)
    ssq = st1[:, 4:, :].reshape(NW, h).sum(0)
    mean1 = ssum / e
    var1 = ssq / e - mean1 * mean1
    s1 = g1 / jnp.sqrt(var1 + 1e-5)
    t1 = be1 - mean1 * s1

    z2, st2 = _tc_mlp_mid(z1, s1.reshape(1, -1), t1.reshape(1, -1), eW2,
                          eb2.reshape(1, -1), e, h, eW2.shape[1])

    mean2 = st2[0] / e
    var2 = st2[1] / e - mean2 * mean2
    s2 = g2 / jnp.sqrt(var2 + 1e-5)
    t2 = be2 - mean2 * s2

    outT = _tc_mlp_out(z2, s2.reshape(1, -1), t2.reshape(1, -1), eW3,
                       eb3.reshape(1, -1), e, eW2.shape[1], eW3.shape[1])
    return outT.T


# speedup vs baseline: 17.9876x; 1.6345x over previous
import functools

import jax
import jax.numpy as jnp
from jax import lax
from jax.experimental import pallas as pl
from jax.experimental.pallas import tpu as pltpu
from jax.experimental.pallas import tpu_sc as plsc

NC = 2
NS = 16
NW = NC * NS

C = 80
NB = 5


def _mesh():
    return plsc.VectorSubcoreMesh(core_axis_name="c", subcore_axis_name="s")


def _deg_partials(ones, dst3, zeros16, n, ch):
    rows = n // NS

    @functools.partial(
        pl.kernel,
        out_type=jax.ShapeDtypeStruct((NC, n, 16), jnp.float32),
        mesh=_mesh(),
        compiler_params=pltpu.CompilerParams(use_tc_tiling_on_sc=False),
        scratch_types=[
            pltpu.VMEM((ch, C), jnp.int32),
            pltpu.VMEM((C, 16), jnp.float32),
            pltpu.VMEM_SHARED((n, 16), jnp.float32),
            pltpu.SemaphoreType.DMA,
        ],
    )
    def k(ones_hbm, dst_hbm, zero_hbm, out_hbm, dst_v, ones_v, deg_sh, sem):
        c = lax.axis_index("c")
        s = lax.axis_index("s")
        wid = s * NC + c
        pltpu.sync_copy(zero_hbm.at[pl.ds(s * rows, rows)],
                        deg_sh.at[pl.ds(s * rows, rows)])
        pltpu.sync_copy(dst_hbm.at[wid], dst_v)
        pltpu.sync_copy(ones_hbm, ones_v)
        plsc.subcore_barrier()

        def body(j, carry):
            pltpu.async_copy(ones_v, deg_sh.at[dst_v.at[j]], sem, add=True)
            return carry

        lax.fori_loop(0, ch, body, 0)

        def drain(j, carry):
            pltpu.make_async_copy(ones_v, deg_sh.at[dst_v.at[0]], sem).wait()
            return carry

        lax.fori_loop(0, ch, drain, 0)
        plsc.subcore_barrier()
        pltpu.sync_copy(deg_sh.at[pl.ds(s * rows, rows)],
                        out_hbm.at[c, pl.ds(s * rows, rows)])

    return k(ones, dst3, zeros16)


def _seg_sum(u, src3, dst3, zeros64, n, h, ch):
    rows = n // NS

    @functools.partial(
        pl.kernel,
        out_type=jax.ShapeDtypeStruct((NC, n, h), jnp.float32),
        mesh=_mesh(),
        compiler_params=pltpu.CompilerParams(use_tc_tiling_on_sc=False),
        scratch_types=[
            pltpu.VMEM((ch, C), jnp.int32),
            pltpu.VMEM((ch, C), jnp.int32),
            pltpu.VMEM((NB, C, h), jnp.float32),
            pltpu.VMEM_SHARED((n, h), jnp.float32),
            pltpu.SemaphoreType.DMA((NB,)),
            pltpu.SemaphoreType.DMA((NB,)),
        ],
    )
    def k(u_hbm, src_hbm, dst_hbm, zero_hbm, out_hbm, src_v, dst_v, gath_v,
          acc_sh, sem, ssem):
        c = lax.axis_index("c")
        s = lax.axis_index("s")
        wid = s * NC + c
        pltpu.sync_copy(zero_hbm.at[pl.ds(s * rows, rows)],
                        acc_sh.at[pl.ds(s * rows, rows)])
        pltpu.sync_copy(src_hbm.at[wid], src_v)
        pltpu.sync_copy(dst_hbm.at[wid], dst_v)
        plsc.subcore_barrier()

        LA = 2
        for b in range(LA):
            pltpu.async_copy(u_hbm.at[src_v.at[b]], gath_v.at[b], sem.at[b])

        def body(i, carry):
            j = i * NB
            for b in range(NB):
                jj = j + b
                bn = (b + LA) % NB
                nxt = jj + LA

                @pl.when(nxt < ch)
                def _():
                    @pl.when(nxt >= NB)
                    def _():
                        pltpu.make_async_copy(
                            gath_v.at[bn], acc_sh.at[dst_v.at[0]],
                            ssem.at[bn]).wait()

                    pltpu.async_copy(u_hbm.at[src_v.at[nxt]], gath_v.at[bn],
                                     sem.at[bn])

                pltpu.make_async_copy(u_hbm.at[src_v.at[jj]], gath_v.at[b],
                                      sem.at[b]).wait()
                pltpu.async_copy(gath_v.at[b], acc_sh.at[dst_v.at[jj]],
                                 ssem.at[b], add=True)
            return carry

        lax.fori_loop(0, ch // NB, body, 0)
        for b in range(NB):
            pltpu.make_async_copy(gath_v.at[b], acc_sh.at[dst_v.at[0]],
                                  ssem.at[b]).wait()
        plsc.subcore_barrier()
        pltpu.sync_copy(acc_sh.at[pl.ds(s * rows, rows)],
                        out_hbm.at[c, pl.ds(s * rows, rows)])

    return k(u, src3, dst3, zeros64)


def _edge_head(p, q, src3, dst3, n, h, e, ch):
    ew = e // NW

    @functools.partial(
        pl.kernel,
        out_type=(
            jax.ShapeDtypeStruct((e, h), jnp.float32),
            jax.ShapeDtypeStruct((NW, 8, 16), jnp.float32),
        ),
        mesh=_mesh(),
        compiler_params=pltpu.CompilerParams(use_tc_tiling_on_sc=False),
        scratch_types=[
            pltpu.VMEM((ch, C), jnp.int32),
            pltpu.VMEM((ch, C), jnp.int32),
            pltpu.VMEM((NB, C, h), jnp.float32),
            pltpu.VMEM((NB, C, h), jnp.float32),
            pltpu.VMEM((8, 16), jnp.float32),
            pltpu.SemaphoreType.DMA((NB,)),
            pltpu.SemaphoreType.DMA((NB,)),
            pltpu.SemaphoreType.DMA((NB,)),
        ],
    )
    def k(p_hbm, q_hbm, src_hbm, dst_hbm, z1_hbm, st_hbm, src_v, dst_v, a_v,
          b_v, st_v, sga, sgb, sw):
        c = lax.axis_index("c")
        s = lax.axis_index("s")
        wid = s * NC + c
        pltpu.sync_copy(src_hbm.at[wid], src_v)
        pltpu.sync_copy(dst_hbm.at[wid], dst_v)
        zero = jnp.zeros((16,), jnp.float32)
        LA = 2

        for b in range(LA):
            pltpu.async_copy(p_hbm.at[src_v.at[b]], a_v.at[b], sga.at[b])
            pltpu.async_copy(q_hbm.at[dst_v.at[b]], b_v.at[b], sgb.at[b])

        def outer(i, carry):
            j = i * NB
            for b in range(NB):
                jj = j + b
                bn = (b + LA) % NB
                nxt = jj + LA

                @pl.when(nxt < ch)
                def _():
                    @pl.when(nxt >= NB)
                    def _():
                        pltpu.make_async_copy(
                            a_v.at[bn], z1_hbm.at[pl.ds(0, C)],
                            sw.at[bn]).wait()

                    pltpu.async_copy(p_hbm.at[src_v.at[nxt]], a_v.at[bn],
                                     sga.at[bn])
                    pltpu.async_copy(q_hbm.at[dst_v.at[nxt]], b_v.at[bn],
                                     sgb.at[bn])

                pltpu.make_async_copy(p_hbm.at[src_v.at[jj]], a_v.at[b],
                                      sga.at[b]).wait()
                pltpu.make_async_copy(q_hbm.at[dst_v.at[jj]], b_v.at[b],
                                      sgb.at[b]).wait()

                def row(r, cr):
                    acc = list(cr)
                    for kk in range(4):
                        sl = pl.ds(kk * 16, 16)
                        z = jnp.maximum(a_v[b, r, sl] + b_v[b, r, sl], 0.0)
                        a_v[b, r, sl] = z
                        acc[kk] = acc[kk] + z
                        acc[4 + kk] = acc[4 + kk] + z * z
                    return tuple(acc)

                carry = lax.fori_loop(0, C, row, carry)
                pltpu.async_copy(a_v.at[b],
                                 z1_hbm.at[pl.ds(wid * ew + jj * C, C)],
                                 sw.at[b])
            return carry

        carry = lax.fori_loop(0, ch // NB, outer, (zero,) * 8)
        for b in range(NB):
            pltpu.make_async_copy(a_v.at[b], z1_hbm.at[pl.ds(0, C)],
                                  sw.at[b]).wait()
        for kk in range(8):
            st_v[kk, :] = carry[kk]
        pltpu.sync_copy(st_v, st_hbm.at[wid])

    return k(p, q, src3, dst3)


def _tc_in(x, w1, degp, n, d, h):
    bn = 2560

    def body(x_b, w_r, deg_b, g_b, u_b, di_b):
        deg = deg_b[...]
        dtot = deg[0, :, 0:1] + deg[1, :, 0:1] + 1.0
        di = lax.rsqrt(dtot)
        g = jnp.dot(x_b[...], w_r[...], preferred_element_type=jnp.float32)
        g_b[...] = g
        u_b[...] = g * di
        di_b[...] = jnp.broadcast_to(di, di_b.shape)

    return pl.pallas_call(
        body,
        grid=(n // bn,),
        in_specs=[
            pl.BlockSpec((bn, d), lambda i: (i, 0)),
            pl.BlockSpec((d, h), lambda i: (0, 0)),
            pl.BlockSpec((NC, bn, 16), lambda i: (0, i, 0)),
        ],
        out_specs=[
            pl.BlockSpec((bn, h), lambda i: (i, 0)),
            pl.BlockSpec((bn, h), lambda i: (i, 0)),
            pl.BlockSpec((bn, 16), lambda i: (i, 0)),
        ],
        out_shape=[
            jax.ShapeDtypeStruct((n, h), jnp.float32),
            jax.ShapeDtypeStruct((n, h), jnp.float32),
            jax.ShapeDtypeStruct((n, 16), jnp.float32),
        ],
    )(x, w1, degp)


def _tc_conv(acc, g, di, b, w, n, h):
    bn = 2560

    def body(acc_b, g_b, di_b, b_r, w_r, gn_b, un_b):
        a = acc_b[...]
        dv = di_b[...][:, 0:1]
        hh = jnp.maximum(dv * (a[0] + a[1]) + dv * dv * g_b[...] + b_r[...],
                         0.0)
        gn = jnp.dot(hh, w_r[...], preferred_element_type=jnp.float32)
        gn_b[...] = gn
        un_b[...] = gn * dv

    return pl.pallas_call(
        body,
        grid=(n // bn,),
        in_specs=[
            pl.BlockSpec((NC, bn, h), lambda i: (0, i, 0)),
            pl.BlockSpec((bn, h), lambda i: (i, 0)),
            pl.BlockSpec((bn, 16), lambda i: (i, 0)),
            pl.BlockSpec((1, h), lambda i: (0, 0)),
            pl.BlockSpec((h, h), lambda i: (0, 0)),
        ],
        out_specs=[
            pl.BlockSpec((bn, h), lambda i: (i, 0)),
            pl.BlockSpec((bn, h), lambda i: (i, 0)),
        ],
        out_shape=[
            jax.ShapeDtypeStruct((n, h), jnp.float32),
            jax.ShapeDtypeStruct((n, h), jnp.float32),
        ],
    )(acc, g, di, b, w)


def _tc_pq(acc, g, di, b, ea, eb, eb1, n, h):
    bn = 2560

    def body(acc_b, g_b, di_b, b_r, ea_r, eb_r, eb1_r, p_b, q_b):
        a = acc_b[...]
        dv = di_b[...][:, 0:1]
        h3 = dv * (a[0] + a[1]) + dv * dv * g_b[...] + b_r[...]
        p_b[...] = (jnp.dot(h3, ea_r[...], preferred_element_type=jnp.float32)
                    + eb1_r[...])
        q_b[...] = jnp.dot(h3, eb_r[...], preferred_element_type=jnp.float32)

    return pl.pallas_call(
        body,
        grid=(n // bn,),
        in_specs=[
            pl.BlockSpec((NC, bn, h), lambda i: (0, i, 0)),
            pl.BlockSpec((bn, h), lambda i: (i, 0)),
            pl.BlockSpec((bn, 16), lambda i: (i, 0)),
            pl.BlockSpec((1, h), lambda i: (0, 0)),
            pl.BlockSpec((h, h), lambda i: (0, 0)),
            pl.BlockSpec((h, h), lambda i: (0, 0)),
            pl.BlockSpec((1, h), lambda i: (0, 0)),
        ],
        out_specs=[
            pl.BlockSpec((bn, h), lambda i: (i, 0)),
            pl.BlockSpec((bn, h), lambda i: (i, 0)),
        ],
        out_shape=[
            jax.ShapeDtypeStruct((n, h), jnp.float32),
            jax.ShapeDtypeStruct((n, h), jnp.float32),
        ],
    )(acc, g, di, b, ea, eb, eb1)


def _tc_mlp_mid(z1, s, t, w, b, e, hin, hout):
    be = 6400

    def body(z_b, s_r, t_r, w_r, b_r, z2_b, st_b):
        zn = z_b[...] * s_r[...] + t_r[...]
        z2 = jnp.maximum(
            jnp.dot(zn, w_r[...], preferred_element_type=jnp.float32)
            + b_r[...], 0.0)
        z2_b[...] = z2
        st = jnp.concatenate(
            [jnp.sum(z2, axis=0, keepdims=True),
             jnp.sum(z2 * z2, axis=0, keepdims=True)], axis=0)

        @pl.when(pl.program_id(0) == 0)
        def _():
            st_b[...] = st

        @pl.when(pl.program_id(0) != 0)
        def _():
            st_b[...] = st_b[...] + st

    return pl.pallas_call(
        body,
        grid=(e // be,),
        in_specs=[
            pl.BlockSpec((be, hin), lambda i: (i, 0)),
            pl.BlockSpec((1, hin), lambda i: (0, 0)),
            pl.BlockSpec((1, hin), lambda i: (0, 0)),
            pl.BlockSpec((hin, hout), lambda i: (0, 0)),
            pl.BlockSpec((1, hout), lambda i: (0, 0)),
        ],
        out_specs=[
            pl.BlockSpec((be, hout), lambda i: (i, 0)),
            pl.BlockSpec((2, hout), lambda i: (0, 0)),
        ],
        out_shape=[
            jax.ShapeDtypeStruct((e, hout), jnp.float32),
            jax.ShapeDtypeStruct((2, hout), jnp.float32),
        ],
    )(z1, s, t, w, b)


def _tc_mlp_out(z2, s, t, w, b, e, hin, hout):
    be = 6400

    def body(z_b, s_r, t_r, w_r, b_r, o_b):
        zn = z_b[...] * s_r[...] + t_r[...]
        o = (jnp.dot(zn, w_r[...],
                     preferred_element_type=jnp.float32) + b_r[...])
        o_b[...] = o.T

    return pl.pallas_call(
        body,
        grid=(e // be,),
        in_specs=[
            pl.BlockSpec((be, hin), lambda i: (i, 0)),
            pl.BlockSpec((1, hin), lambda i: (0, 0)),
            pl.BlockSpec((1, hin), lambda i: (0, 0)),
            pl.BlockSpec((hin, hout), lambda i: (0, 0)),
            pl.BlockSpec((1, hout), lambda i: (0, 0)),
        ],
        out_specs=pl.BlockSpec((hout, be), lambda i: (0, i)),
        out_shape=jax.ShapeDtypeStruct((hout, e), jnp.float32),
    )(z2, s, t, w, b)


def kernel(x, edge_index, W1, b1, W2, b2, W3, b3, eW1, eb1, g1, be1, eW2,
           eb2, g2, be2, eW3, eb3):
    n, d = x.shape
    e = edge_index.shape[1]
    h = W1.shape[1]
    ew = e // NW
    ch = ew // C
    np_ = ((n + 2559) // 2560) * 2560

    src3 = edge_index[0].reshape(NW, ch, C)
    dst3 = edge_index[1].reshape(NW, ch, C)
    zeros64 = jnp.zeros((np_, h), jnp.float32)
    zeros16 = jnp.zeros((np_, 16), jnp.float32)
    ones = jnp.ones((C, 16), jnp.float32)
    xp = jnp.pad(x, ((0, np_ - n), (0, 0)))

    degp = _deg_partials(ones, dst3, zeros16, np_, ch)
    g_1, u_1, di = _tc_in(xp, W1, degp, np_, d, h)
    acc1 = _seg_sum(u_1, src3, dst3, zeros64, np_, h, ch)
    g_2, u_2 = _tc_conv(acc1, g_1, di, b1.reshape(1, h), W2, np_, h)
    acc2 = _seg_sum(u_2, src3, dst3, zeros64, np_, h, ch)
    g_3, u_3 = _tc_conv(acc2, g_2, di, b2.reshape(1, h), W3, np_, h)
    acc3 = _seg_sum(u_3, src3, dst3, zeros64, np_, h, ch)
    P, Q = _tc_pq(acc3, g_3, di, b3.reshape(1, h), eW1[:h], eW1[h:],
                  eb1.reshape(1, h), np_, h)

    z1, st1 = _edge_head(P, Q, src3, dst3, np_, h, e, ch)

    ssum = st1[:, :4, :].reshape(NW, h).sum(0)
    ssq = st1[:, 4:, :].reshape(NW, h).sum(0)
    mean1 = ssum / e
    var1 = ssq / e - mean1 * mean1
    s1 = g1 / jnp.sqrt(var1 + 1e-5)
    t1 = be1 - mean1 * s1

    z2, st2 = _tc_mlp_mid(z1, s1.reshape(1, -1), t1.reshape(1, -1), eW2,
                          eb2.reshape(1, -1), e, h, eW2.shape[1])

    mean2 = st2[0] / e
    var2 = st2[1] / e - mean2 * mean2
    s2 = g2 / jnp.sqrt(var2 + 1e-5)
    t2 = be2 - mean2 * s2

    outT = _tc_mlp_out(z2, s2.reshape(1, -1), t2.reshape(1, -1), eW3,
                       eb3.reshape(1, -1), e, eW2.shape[1], eW3.shape[1])
    return outT.T


# speedup vs baseline: 19.3420x; 1.0753x over previous
import functools

import jax
import jax.numpy as jnp
from jax import lax
from jax.experimental import pallas as pl
from jax.experimental.pallas import tpu as pltpu
from jax.experimental.pallas import tpu_sc as plsc

NC = 2
NS = 16
NW = NC * NS

C = 80
NB = 5


def _mesh():
    return plsc.VectorSubcoreMesh(core_axis_name="c", subcore_axis_name="s")


def _deg_partials(ones, dst3, zeros16, n, ch):
    rows = n // NS

    @functools.partial(
        pl.kernel,
        out_type=jax.ShapeDtypeStruct((NC, n, 16), jnp.float32),
        mesh=_mesh(),
        compiler_params=pltpu.CompilerParams(use_tc_tiling_on_sc=False),
        scratch_types=[
            pltpu.VMEM((ch, C), jnp.int32),
            pltpu.VMEM((C, 16), jnp.float32),
            pltpu.VMEM_SHARED((n, 16), jnp.float32),
            pltpu.SemaphoreType.DMA,
        ],
    )
    def k(ones_hbm, dst_hbm, zero_hbm, out_hbm, dst_v, ones_v, deg_sh, sem):
        c = lax.axis_index("c")
        s = lax.axis_index("s")
        wid = s * NC + c
        pltpu.sync_copy(zero_hbm.at[pl.ds(s * rows, rows)],
                        deg_sh.at[pl.ds(s * rows, rows)])
        pltpu.sync_copy(dst_hbm.at[wid], dst_v)
        pltpu.sync_copy(ones_hbm, ones_v)
        plsc.subcore_barrier()

        def body(j, carry):
            pltpu.async_copy(ones_v, deg_sh.at[dst_v.at[j]], sem, add=True)
            return carry

        lax.fori_loop(0, ch, body, 0)

        def drain(j, carry):
            pltpu.make_async_copy(ones_v, deg_sh.at[dst_v.at[0]], sem).wait()
            return carry

        lax.fori_loop(0, ch, drain, 0)
        plsc.subcore_barrier()
        pltpu.sync_copy(deg_sh.at[pl.ds(s * rows, rows)],
                        out_hbm.at[c, pl.ds(s * rows, rows)])

    return k(ones, dst3, zeros16)


def _seg_sum(u, src3, dst3, zeros64, n, h, ch):
    rows = n // NS

    @functools.partial(
        pl.kernel,
        out_type=jax.ShapeDtypeStruct((NC, n, h), jnp.float32),
        mesh=_mesh(),
        compiler_params=pltpu.CompilerParams(use_tc_tiling_on_sc=False),
        scratch_types=[
            pltpu.VMEM((ch, C), jnp.int32),
            pltpu.VMEM((ch, C), jnp.int32),
            pltpu.VMEM((NB, C, h), jnp.float32),
            pltpu.VMEM_SHARED((n, h), jnp.float32),
            pltpu.SemaphoreType.DMA((NB,)),
            pltpu.SemaphoreType.DMA((NB,)),
        ],
    )
    def k(u_hbm, src_hbm, dst_hbm, zero_hbm, out_hbm, src_v, dst_v, gath_v,
          acc_sh, sem, ssem):
        c = lax.axis_index("c")
        s = lax.axis_index("s")
        wid = s * NC + c
        pltpu.sync_copy(zero_hbm.at[pl.ds(s * rows, rows)],
                        acc_sh.at[pl.ds(s * rows, rows)])
        pltpu.sync_copy(src_hbm.at[wid], src_v)
        pltpu.sync_copy(dst_hbm.at[wid], dst_v)
        plsc.subcore_barrier()

        LA = 2
        for b in range(LA):
            pltpu.async_copy(u_hbm.at[src_v.at[b]], gath_v.at[b], sem.at[b])

        def body(i, carry):
            j = i * NB
            for b in range(NB):
                jj = j + b
                bn = (b + LA) % NB
                nxt = jj + LA

                @pl.when(nxt < ch)
                def _():
                    @pl.when(nxt >= NB)
                    def _():
                        pltpu.make_async_copy(
                            gath_v.at[bn], acc_sh.at[dst_v.at[0]],
                            ssem.at[bn]).wait()

                    pltpu.async_copy(u_hbm.at[src_v.at[nxt]], gath_v.at[bn],
                                     sem.at[bn])

                pltpu.make_async_copy(u_hbm.at[src_v.at[jj]], gath_v.at[b],
                                      sem.at[b]).wait()
                pltpu.async_copy(gath_v.at[b], acc_sh.at[dst_v.at[jj]],
                                 ssem.at[b], add=True)
            return carry

        lax.fori_loop(0, ch // NB, body, 0)
        for b in range(NB):
            pltpu.make_async_copy(gath_v.at[b], acc_sh.at[dst_v.at[0]],
                                  ssem.at[b]).wait()
        plsc.subcore_barrier()
        pltpu.sync_copy(acc_sh.at[pl.ds(s * rows, rows)],
                        out_hbm.at[c, pl.ds(s * rows, rows)])

    return k(u, src3, dst3, zeros64)


def _edge_head(p, q, idx4, n, h, e, ch):
    e2 = e // 2
    c2 = C // 2
    ew2 = e2 // NW

    @functools.partial(
        pl.kernel,
        out_type=(
            jax.ShapeDtypeStruct((e2, 2 * h), jnp.float32),
            jax.ShapeDtypeStruct((NW, 8, 16), jnp.float32),
        ),
        mesh=_mesh(),
        compiler_params=pltpu.CompilerParams(use_tc_tiling_on_sc=False),
        scratch_types=[
            pltpu.VMEM((4, ch, c2), jnp.int32),
            pltpu.VMEM((4, NB, c2, h), jnp.float32),
            pltpu.VMEM((NB, c2, 2 * h), jnp.float32),
            pltpu.VMEM((8, 16), jnp.float32),
            pltpu.SemaphoreType.DMA((4, NB)),
            pltpu.SemaphoreType.DMA((NB,)),
        ],
    )
    def k(p_hbm, q_hbm, idx_hbm, z1_hbm, st_hbm, idx_v, g_v, z_v, st_v, sg,
          sw):
        c = lax.axis_index("c")
        s = lax.axis_index("s")
        wid = s * NC + c
        for t in range(4):
            pltpu.sync_copy(idx_hbm.at[t, wid], idx_v.at[t])
        zero = jnp.zeros((16,), jnp.float32)
        LA = 2

        def fire(jj, b):
            pltpu.async_copy(p_hbm.at[idx_v.at[0, jj]], g_v.at[0, b],
                             sg.at[0, b])
            pltpu.async_copy(q_hbm.at[idx_v.at[1, jj]], g_v.at[1, b],
                             sg.at[1, b])
            pltpu.async_copy(p_hbm.at[idx_v.at[2, jj]], g_v.at[2, b],
                             sg.at[2, b])
            pltpu.async_copy(q_hbm.at[idx_v.at[3, jj]], g_v.at[3, b],
                             sg.at[3, b])

        def wait_g(jj, b):
            for t in range(4):
                src = p_hbm if t in (0, 2) else q_hbm
                pltpu.make_async_copy(src.at[idx_v.at[t, jj]], g_v.at[t, b],
                                      sg.at[t, b]).wait()

        for b in range(LA):
            fire(b, b)

        def outer(i, carry):
            j = i * NB
            for b in range(NB):
                jj = j + b
                bn = (b + LA) % NB
                nxt = jj + LA

                @pl.when(nxt < ch)
                def _():
                    @pl.when(nxt >= NB)
                    def _():
                        pltpu.make_async_copy(
                            z_v.at[bn], z1_hbm.at[pl.ds(0, c2)],
                            sw.at[bn]).wait()

                    fire(nxt, bn)

                wait_g(jj, b)

                def row(r, cr):
                    acc = list(cr)
                    for kk in range(4):
                        sl = pl.ds(kk * 16, 16)
                        zl = jnp.maximum(g_v[0, b, r, sl] + g_v[1, b, r, sl],
                                         0.0)
                        z_v[b, r, sl] = zl
                        zh = jnp.maximum(g_v[2, b, r, sl] + g_v[3, b, r, sl],
                                         0.0)
                        z_v[b, r, pl.ds(h + kk * 16, 16)] = zh
                        acc[kk] = acc[kk] + zl + zh
                        acc[4 + kk] = acc[4 + kk] + zl * zl + zh * zh
                    return tuple(acc)

                carry = lax.fori_loop(0, c2, row, carry)
                pltpu.async_copy(z_v.at[b],
                                 z1_hbm.at[pl.ds(wid * ew2 + jj * c2, c2)],
                                 sw.at[b])
            return carry

        carry = lax.fori_loop(0, ch // NB, outer, (zero,) * 8)
        for b in range(NB):
            pltpu.make_async_copy(z_v.at[b], z1_hbm.at[pl.ds(0, c2)],
                                  sw.at[b]).wait()
        for kk in range(8):
            st_v[kk, :] = carry[kk]
        pltpu.sync_copy(st_v, st_hbm.at[wid])

    return k(p, q, idx4)


def _tc_in(x, w1, degp, n, d, h):
    bn = 2560

    def body(x_b, w_r, deg_b, g_b, u_b, di_b):
        deg = deg_b[...]
        dtot = deg[0, :, 0:1] + deg[1, :, 0:1] + 1.0
        di = lax.rsqrt(dtot)
        g = jnp.dot(x_b[...], w_r[...], preferred_element_type=jnp.float32)
        g_b[...] = g
        u_b[...] = g * di
        di_b[...] = jnp.broadcast_to(di, di_b.shape)

    return pl.pallas_call(
        body,
        grid=(n // bn,),
        in_specs=[
            pl.BlockSpec((bn, d), lambda i: (i, 0)),
            pl.BlockSpec((d, h), lambda i: (0, 0)),
            pl.BlockSpec((NC, bn, 16), lambda i: (0, i, 0)),
        ],
        out_specs=[
            pl.BlockSpec((bn, h), lambda i: (i, 0)),
            pl.BlockSpec((bn, h), lambda i: (i, 0)),
            pl.BlockSpec((bn, 16), lambda i: (i, 0)),
        ],
        out_shape=[
            jax.ShapeDtypeStruct((n, h), jnp.float32),
            jax.ShapeDtypeStruct((n, h), jnp.float32),
            jax.ShapeDtypeStruct((n, 16), jnp.float32),
        ],
    )(x, w1, degp)


def _tc_conv(acc, g, di, b, w, n, h):
    bn = 2560

    def body(acc_b, g_b, di_b, b_r, w_r, gn_b, un_b):
        a = acc_b[...]
        dv = di_b[...][:, 0:1]
        hh = jnp.maximum(dv * (a[0] + a[1]) + dv * dv * g_b[...] + b_r[...],
                         0.0)
        gn = jnp.dot(hh, w_r[...], preferred_element_type=jnp.float32)
        gn_b[...] = gn
        un_b[...] = gn * dv

    return pl.pallas_call(
        body,
        grid=(n // bn,),
        in_specs=[
            pl.BlockSpec((NC, bn, h), lambda i: (0, i, 0)),
            pl.BlockSpec((bn, h), lambda i: (i, 0)),
            pl.BlockSpec((bn, 16), lambda i: (i, 0)),
            pl.BlockSpec((1, h), lambda i: (0, 0)),
            pl.BlockSpec((h, h), lambda i: (0, 0)),
        ],
        out_specs=[
            pl.BlockSpec((bn, h), lambda i: (i, 0)),
            pl.BlockSpec((bn, h), lambda i: (i, 0)),
        ],
        out_shape=[
            jax.ShapeDtypeStruct((n, h), jnp.float32),
            jax.ShapeDtypeStruct((n, h), jnp.float32),
        ],
    )(acc, g, di, b, w)


def _tc_pq(acc, g, di, b, ea, eb, eb1, n, h):
    bn = 2560

    def body(acc_b, g_b, di_b, b_r, ea_r, eb_r, eb1_r, p_b, q_b):
        a = acc_b[...]
        dv = di_b[...][:, 0:1]
        h3 = dv * (a[0] + a[1]) + dv * dv * g_b[...] + b_r[...]
        p_b[...] = (jnp.dot(h3, ea_r[...], preferred_element_type=jnp.float32)
                    + eb1_r[...])
        q_b[...] = jnp.dot(h3, eb_r[...], preferred_element_type=jnp.float32)

    return pl.pallas_call(
        body,
        grid=(n // bn,),
        in_specs=[
            pl.BlockSpec((NC, bn, h), lambda i: (0, i, 0)),
            pl.BlockSpec((bn, h), lambda i: (i, 0)),
            pl.BlockSpec((bn, 16), lambda i: (i, 0)),
            pl.BlockSpec((1, h), lambda i: (0, 0)),
            pl.BlockSpec((h, h), lambda i: (0, 0)),
            pl.BlockSpec((h, h), lambda i: (0, 0)),
            pl.BlockSpec((1, h), lambda i: (0, 0)),
        ],
        out_specs=[
            pl.BlockSpec((bn, h), lambda i: (i, 0)),
            pl.BlockSpec((bn, h), lambda i: (i, 0)),
        ],
        out_shape=[
            jax.ShapeDtypeStruct((n, h), jnp.float32),
            jax.ShapeDtypeStruct((n, h), jnp.float32),
        ],
    )(acc, g, di, b, ea, eb, eb1)


def _tc_mlp_mid(z1p, s, t, w, b, e, hin, hout):
    e2 = e // 2
    be = 6400
    nb = e2 // be

    def body(z_b, s_r, t_r, w_r, b_r, z2_b, st_b):
        zn = z_b[...] * s_r[...] + t_r[...]
        z2 = jnp.maximum(
            jnp.dot(zn, w_r[...], preferred_element_type=jnp.float32)
            + b_r[...], 0.0)
        z2_b[...] = z2
        st = jnp.concatenate(
            [jnp.sum(z2, axis=0, keepdims=True),
             jnp.sum(z2 * z2, axis=0, keepdims=True)], axis=0)

        @pl.when(pl.program_id(0) == 0)
        def _():
            st_b[...] = st

        @pl.when(pl.program_id(0) != 0)
        def _():
            st_b[...] = st_b[...] + st

    return pl.pallas_call(
        body,
        grid=(nb,),
        in_specs=[
            pl.BlockSpec((be, 2 * hin), lambda i: (i, 0)),
            pl.BlockSpec((1, 2 * hin), lambda i: (0, 0)),
            pl.BlockSpec((1, 2 * hin), lambda i: (0, 0)),
            pl.BlockSpec((2 * hin, 2 * hout), lambda i: (0, 0)),
            pl.BlockSpec((1, 2 * hout), lambda i: (0, 0)),
        ],
        out_specs=[
            pl.BlockSpec((be, 2 * hout), lambda i: (i, 0)),
            pl.BlockSpec((2, 2 * hout), lambda i: (0, 0)),
        ],
        out_shape=[
            jax.ShapeDtypeStruct((e2, 2 * hout), jnp.float32),
            jax.ShapeDtypeStruct((2, 2 * hout), jnp.float32),
        ],
    )(z1p, s, t, w, b)


def _tc_mlp_out(z2p, s, t, w, b, e, hin, hout):
    e2 = e // 2
    be = 6400
    nb = e2 // be

    def body(z_b, s_r, t_r, w_r, b_r, ol_b, oh_b):
        zn = z_b[...] * s_r[...] + t_r[...]
        o = (jnp.dot(zn, w_r[...],
                     preferred_element_type=jnp.float32) + b_r[...])
        ot = o.T
        ol_b[...] = ot[:hout]
        oh_b[...] = ot[hout:]

    return pl.pallas_call(
        body,
        grid=(nb,),
        in_specs=[
            pl.BlockSpec((be, 2 * hin), lambda i: (i, 0)),
            pl.BlockSpec((1, 2 * hin), lambda i: (0, 0)),
            pl.BlockSpec((1, 2 * hin), lambda i: (0, 0)),
            pl.BlockSpec((2 * hin, 2 * hout), lambda i: (0, 0)),
            pl.BlockSpec((1, 2 * hout), lambda i: (0, 0)),
        ],
        out_specs=[
            pl.BlockSpec((hout, be), lambda i: (0, i)),
            pl.BlockSpec((hout, be), lambda i: (0, i)),
        ],
        out_shape=[
            jax.ShapeDtypeStruct((hout, e2), jnp.float32),
            jax.ShapeDtypeStruct((hout, e2), jnp.float32),
        ],
    )(z2p, s, t, w, b)


def kernel(x, edge_index, W1, b1, W2, b2, W3, b3, eW1, eb1, g1, be1, eW2,
           eb2, g2, be2, eW3, eb3):
    n, d = x.shape
    e = edge_index.shape[1]
    h = W1.shape[1]
    ew = e // NW
    ch = ew // C
    np_ = ((n + 2559) // 2560) * 2560

    src3 = edge_index[0].reshape(NW, ch, C)
    dst3 = edge_index[1].reshape(NW, ch, C)
    zeros64 = jnp.zeros((np_, h), jnp.float32)
    zeros16 = jnp.zeros((np_, 16), jnp.float32)
    ones = jnp.ones((C, 16), jnp.float32)
    xp = jnp.pad(x, ((0, np_ - n), (0, 0)))

    degp = _deg_partials(ones, dst3, zeros16, np_, ch)
    g_1, u_1, di = _tc_in(xp, W1, degp, np_, d, h)
    acc1 = _seg_sum(u_1, src3, dst3, zeros64, np_, h, ch)
    g_2, u_2 = _tc_conv(acc1, g_1, di, b1.reshape(1, h), W2, np_, h)
    acc2 = _seg_sum(u_2, src3, dst3, zeros64, np_, h, ch)
    g_3, u_3 = _tc_conv(acc2, g_2, di, b2.reshape(1, h), W3, np_, h)
    acc3 = _seg_sum(u_3, src3, dst3, zeros64, np_, h, ch)
    P, Q = _tc_pq(acc3, g_3, di, b3.reshape(1, h), eW1[:h], eW1[h:],
                  eb1.reshape(1, h), np_, h)

    e2 = e // 2
    c2 = C // 2
    idx4 = jnp.stack([
        edge_index[0, :e2].reshape(NW, ch, c2),
        edge_index[1, :e2].reshape(NW, ch, c2),
        edge_index[0, e2:].reshape(NW, ch, c2),
        edge_index[1, e2:].reshape(NW, ch, c2),
    ])
    z1, st1 = _edge_head(P, Q, idx4, np_, h, e, ch)

    ssum = st1[:, :4, :].reshape(NW, h).sum(0)
    ssq = st1[:, 4:, :].reshape(NW, h).sum(0)
    mean1 = ssum / e
    var1 = ssq / e - mean1 * mean1
    s1 = g1 / jnp.sqrt(var1 + 1e-5)
    t1 = be1 - mean1 * s1

    h2 = eW2.shape[1]
    zW2 = jnp.zeros_like(eW2)
    W2p = jnp.concatenate([jnp.concatenate([eW2, zW2], 1),
                           jnp.concatenate([zW2, eW2], 1)], 0)
    z2, st2p = _tc_mlp_mid(z1, jnp.tile(s1, 2).reshape(1, -1),
                           jnp.tile(t1, 2).reshape(1, -1), W2p,
                           jnp.tile(eb2, 2).reshape(1, -1), e, h, h2)

    st2 = st2p[:, :h2] + st2p[:, h2:]
    mean2 = st2[0] / e
    var2 = st2[1] / e - mean2 * mean2
    s2 = g2 / jnp.sqrt(var2 + 1e-5)
    t2 = be2 - mean2 * s2

    h3 = eW3.shape[1]
    zW3 = jnp.zeros_like(eW3)
    W3p = jnp.concatenate([jnp.concatenate([eW3, zW3], 1),
                           jnp.concatenate([zW3, eW3], 1)], 0)
    outTlo, outThi = _tc_mlp_out(z2, jnp.tile(s2, 2).reshape(1, -1),
                                 jnp.tile(t2, 2).reshape(1, -1), W3p,
                                 jnp.tile(eb3, 2).reshape(1, -1), e, h2, h3)
    outT = jnp.concatenate([outTlo, outThi], axis=1)
    return outT.T


# speedup vs baseline: 19.4769x; 1.0070x over previous
import functools

import jax
import jax.numpy as jnp
from jax import lax
from jax.experimental import pallas as pl
from jax.experimental.pallas import tpu as pltpu
from jax.experimental.pallas import tpu_sc as plsc

NC = 2
NS = 16
NW = NC * NS

C = 80
NB = 5


def _mesh():
    return plsc.VectorSubcoreMesh(core_axis_name="c", subcore_axis_name="s")


def _deg_partials(ones, dst3, zeros16, n, ch):
    rows = n // NS

    @functools.partial(
        pl.kernel,
        out_type=jax.ShapeDtypeStruct((NC, n, 16), jnp.float32),
        mesh=_mesh(),
        compiler_params=pltpu.CompilerParams(use_tc_tiling_on_sc=False),
        scratch_types=[
            pltpu.VMEM((ch, C), jnp.int32),
            pltpu.VMEM((C, 16), jnp.float32),
            pltpu.VMEM_SHARED((n, 16), jnp.float32),
            pltpu.SemaphoreType.DMA,
        ],
    )
    def k(ones_hbm, dst_hbm, zero_hbm, out_hbm, dst_v, ones_v, deg_sh, sem):
        c = lax.axis_index("c")
        s = lax.axis_index("s")
        wid = s * NC + c
        pltpu.sync_copy(zero_hbm.at[pl.ds(s * rows, rows)],
                        deg_sh.at[pl.ds(s * rows, rows)])
        pltpu.sync_copy(dst_hbm.at[wid], dst_v)
        pltpu.sync_copy(ones_hbm, ones_v)
        plsc.subcore_barrier()

        def body(j, carry):
            pltpu.async_copy(ones_v, deg_sh.at[dst_v.at[j]], sem, add=True)
            return carry

        lax.fori_loop(0, ch, body, 0)

        def drain(j, carry):
            pltpu.make_async_copy(ones_v, deg_sh.at[dst_v.at[0]], sem).wait()
            return carry

        lax.fori_loop(0, ch, drain, 0)
        plsc.subcore_barrier()
        pltpu.sync_copy(deg_sh.at[pl.ds(s * rows, rows)],
                        out_hbm.at[c, pl.ds(s * rows, rows)])

    return k(ones, dst3, zeros16)


def _seg_sum(u, src3, dst3, zeros64, n, h, ch):
    rows = n // NS

    @functools.partial(
        pl.kernel,
        out_type=jax.ShapeDtypeStruct((NC, n, h), jnp.float32),
        mesh=_mesh(),
        compiler_params=pltpu.CompilerParams(use_tc_tiling_on_sc=False),
        scratch_types=[
            pltpu.VMEM((ch, C), jnp.int32),
            pltpu.VMEM((ch, C), jnp.int32),
            pltpu.VMEM((NB, C, h), jnp.float32),
            pltpu.VMEM_SHARED((n, h), jnp.float32),
            pltpu.SemaphoreType.DMA((NB,)),
            pltpu.SemaphoreType.DMA((NB,)),
        ],
    )
    def k(u_hbm, src_hbm, dst_hbm, zero_hbm, out_hbm, src_v, dst_v, gath_v,
          acc_sh, sem, ssem):
        c = lax.axis_index("c")
        s = lax.axis_index("s")
        wid = s * NC + c
        pltpu.sync_copy(zero_hbm.at[pl.ds(s * rows, rows)],
                        acc_sh.at[pl.ds(s * rows, rows)])
        pltpu.sync_copy(src_hbm.at[wid], src_v)
        pltpu.sync_copy(dst_hbm.at[wid], dst_v)
        plsc.subcore_barrier()

        LA = 2
        for b in range(LA):
            pltpu.async_copy(u_hbm.at[src_v.at[b]], gath_v.at[b], sem.at[b])

        def body(i, carry):
            j = i * NB
            for b in range(NB):
                jj = j + b
                bn = (b + LA) % NB
                nxt = jj + LA

                @pl.when(nxt < ch)
                def _():
                    @pl.when(nxt >= NB)
                    def _():
                        pltpu.make_async_copy(
                            gath_v.at[bn], acc_sh.at[dst_v.at[0]],
                            ssem.at[bn]).wait()

                    pltpu.async_copy(u_hbm.at[src_v.at[nxt]], gath_v.at[bn],
                                     sem.at[bn])

                pltpu.make_async_copy(u_hbm.at[src_v.at[jj]], gath_v.at[b],
                                      sem.at[b]).wait()
                pltpu.async_copy(gath_v.at[b], acc_sh.at[dst_v.at[jj]],
                                 ssem.at[b], add=True)
            return carry

        lax.fori_loop(0, ch // NB, body, 0)
        for b in range(NB):
            pltpu.make_async_copy(gath_v.at[b], acc_sh.at[dst_v.at[0]],
                                  ssem.at[b]).wait()
        plsc.subcore_barrier()
        pltpu.sync_copy(acc_sh.at[pl.ds(s * rows, rows)],
                        out_hbm.at[c, pl.ds(s * rows, rows)])

    return k(u, src3, dst3, zeros64)


def _edge_head(p, q, srcm, dstm, n, h, e, ch):
    e2 = e // 2
    c2 = C // 2
    ew2 = e2 // NW

    @functools.partial(
        pl.kernel,
        out_type=(
            jax.ShapeDtypeStruct((e2, 2 * h), jnp.float32),
            jax.ShapeDtypeStruct((NW, 8, 16), jnp.float32),
        ),
        mesh=_mesh(),
        compiler_params=pltpu.CompilerParams(use_tc_tiling_on_sc=False),
        scratch_types=[
            pltpu.VMEM((ch, C), jnp.int32),
            pltpu.VMEM((ch, C), jnp.int32),
            pltpu.VMEM((NB, C, h), jnp.float32),
            pltpu.VMEM((NB, C, h), jnp.float32),
            pltpu.VMEM((NB, c2, 2 * h), jnp.float32),
            pltpu.VMEM((8, 16), jnp.float32),
            pltpu.SemaphoreType.DMA((NB,)),
            pltpu.SemaphoreType.DMA((NB,)),
            pltpu.SemaphoreType.DMA((NB,)),
        ],
    )
    def k(p_hbm, q_hbm, src_hbm, dst_hbm, z1_hbm, st_hbm, src_v, dst_v, a_v,
          b_v, z_v, st_v, sga, sgb, sw):
        c = lax.axis_index("c")
        s = lax.axis_index("s")
        wid = s * NC + c
        pltpu.sync_copy(src_hbm.at[wid], src_v)
        pltpu.sync_copy(dst_hbm.at[wid], dst_v)
        zero = jnp.zeros((16,), jnp.float32)
        LA = 2

        for b in range(LA):
            pltpu.async_copy(p_hbm.at[src_v.at[b]], a_v.at[b], sga.at[b])
            pltpu.async_copy(q_hbm.at[dst_v.at[b]], b_v.at[b], sgb.at[b])

        def outer(i, carry):
            j = i * NB
            for b in range(NB):
                jj = j + b
                bn = (b + LA) % NB
                nxt = jj + LA

                @pl.when(nxt < ch)
                def _():
                    @pl.when(nxt >= NB)
                    def _():
                        pltpu.make_async_copy(
                            z_v.at[bn], z1_hbm.at[pl.ds(0, c2)],
                            sw.at[bn]).wait()

                    pltpu.async_copy(p_hbm.at[src_v.at[nxt]], a_v.at[bn],
                                     sga.at[bn])
                    pltpu.async_copy(q_hbm.at[dst_v.at[nxt]], b_v.at[bn],
                                     sgb.at[bn])

                pltpu.make_async_copy(p_hbm.at[src_v.at[jj]], a_v.at[b],
                                      sga.at[b]).wait()
                pltpu.make_async_copy(q_hbm.at[dst_v.at[jj]], b_v.at[b],
                                      sgb.at[b]).wait()

                def row(r, cr):
                    acc = list(cr)
                    for kk in range(4):
                        sl = pl.ds(kk * 16, 16)
                        zl = jnp.maximum(a_v[b, r, sl] + b_v[b, r, sl], 0.0)
                        z_v[b, r, sl] = zl
                        zh = jnp.maximum(a_v[b, c2 + r, sl]
                                         + b_v[b, c2 + r, sl], 0.0)
                        z_v[b, r, pl.ds(h + kk * 16, 16)] = zh
                        acc[kk] = acc[kk] + zl + zh
                        acc[4 + kk] = acc[4 + kk] + zl * zl + zh * zh
                    return tuple(acc)

                carry = lax.fori_loop(0, c2, row, carry)
                pltpu.async_copy(z_v.at[b],
                                 z1_hbm.at[pl.ds(wid * ew2 + jj * c2, c2)],
                                 sw.at[b])
            return carry

        carry = lax.fori_loop(0, ch // NB, outer, (zero,) * 8)
        for b in range(NB):
            pltpu.make_async_copy(z_v.at[b], z1_hbm.at[pl.ds(0, c2)],
                                  sw.at[b]).wait()
        for kk in range(8):
            st_v[kk, :] = carry[kk]
        pltpu.sync_copy(st_v, st_hbm.at[wid])

    return k(p, q, srcm, dstm)


def _tc_in(x, w1, degp, n, d, h):
    bn = 2560

    def body(x_b, w_r, deg_b, g_b, u_b, di_b):
        deg = deg_b[...]
        dtot = deg[0, :, 0:1] + deg[1, :, 0:1] + 1.0
        di = lax.rsqrt(dtot)
        g = jnp.dot(x_b[...], w_r[...], preferred_element_type=jnp.float32)
        g_b[...] = g
        u_b[...] = g * di
        di_b[...] = jnp.broadcast_to(di, di_b.shape)

    return pl.pallas_call(
        body,
        grid=(n // bn,),
        in_specs=[
            pl.BlockSpec((bn, d), lambda i: (i, 0)),
            pl.BlockSpec((d, h), lambda i: (0, 0)),
            pl.BlockSpec((NC, bn, 16), lambda i: (0, i, 0)),
        ],
        out_specs=[
            pl.BlockSpec((bn, h), lambda i: (i, 0)),
            pl.BlockSpec((bn, h), lambda i: (i, 0)),
            pl.BlockSpec((bn, 16), lambda i: (i, 0)),
        ],
        out_shape=[
            jax.ShapeDtypeStruct((n, h), jnp.float32),
            jax.ShapeDtypeStruct((n, h), jnp.float32),
            jax.ShapeDtypeStruct((n, 16), jnp.float32),
        ],
    )(x, w1, degp)


def _tc_conv(acc, g, di, b, w, n, h):
    bn = 2560

    def body(acc_b, g_b, di_b, b_r, w_r, gn_b, un_b):
        a = acc_b[...]
        dv = di_b[...][:, 0:1]
        hh = jnp.maximum(dv * (a[0] + a[1]) + dv * dv * g_b[...] + b_r[...],
                         0.0)
        gn = jnp.dot(hh, w_r[...], preferred_element_type=jnp.float32)
        gn_b[...] = gn
        un_b[...] = gn * dv

    return pl.pallas_call(
        body,
        grid=(n // bn,),
        in_specs=[
            pl.BlockSpec((NC, bn, h), lambda i: (0, i, 0)),
            pl.BlockSpec((bn, h), lambda i: (i, 0)),
            pl.BlockSpec((bn, 16), lambda i: (i, 0)),
            pl.BlockSpec((1, h), lambda i: (0, 0)),
            pl.BlockSpec((h, h), lambda i: (0, 0)),
        ],
        out_specs=[
            pl.BlockSpec((bn, h), lambda i: (i, 0)),
            pl.BlockSpec((bn, h), lambda i: (i, 0)),
        ],
        out_shape=[
            jax.ShapeDtypeStruct((n, h), jnp.float32),
            jax.ShapeDtypeStruct((n, h), jnp.float32),
        ],
    )(acc, g, di, b, w)


def _tc_pq(acc, g, di, b, ea, eb, eb1, n, h):
    bn = 2560

    def body(acc_b, g_b, di_b, b_r, ea_r, eb_r, eb1_r, p_b, q_b):
        a = acc_b[...]
        dv = di_b[...][:, 0:1]
        h3 = dv * (a[0] + a[1]) + dv * dv * g_b[...] + b_r[...]
        p_b[...] = (jnp.dot(h3, ea_r[...], preferred_element_type=jnp.float32)
                    + eb1_r[...])
        q_b[...] = jnp.dot(h3, eb_r[...], preferred_element_type=jnp.float32)

    return pl.pallas_call(
        body,
        grid=(n // bn,),
        in_specs=[
            pl.BlockSpec((NC, bn, h), lambda i: (0, i, 0)),
            pl.BlockSpec((bn, h), lambda i: (i, 0)),
            pl.BlockSpec((bn, 16), lambda i: (i, 0)),
            pl.BlockSpec((1, h), lambda i: (0, 0)),
            pl.BlockSpec((h, h), lambda i: (0, 0)),
            pl.BlockSpec((h, h), lambda i: (0, 0)),
            pl.BlockSpec((1, h), lambda i: (0, 0)),
        ],
        out_specs=[
            pl.BlockSpec((bn, h), lambda i: (i, 0)),
            pl.BlockSpec((bn, h), lambda i: (i, 0)),
        ],
        out_shape=[
            jax.ShapeDtypeStruct((n, h), jnp.float32),
            jax.ShapeDtypeStruct((n, h), jnp.float32),
        ],
    )(acc, g, di, b, ea, eb, eb1)


def _tc_mlp_mid(z1p, s, t, w, b, e, hin, hout):
    e2 = e // 2
    be = 6400
    nb = e2 // be

    def body(z_b, s_r, t_r, w_r, b_r, z2_b, st_b):
        zn = z_b[...] * s_r[...] + t_r[...]
        z2 = jnp.maximum(
            jnp.dot(zn, w_r[...], preferred_element_type=jnp.float32)
            + b_r[...], 0.0)
        z2_b[...] = z2
        st = jnp.concatenate(
            [jnp.sum(z2, axis=0, keepdims=True),
             jnp.sum(z2 * z2, axis=0, keepdims=True)], axis=0)

        @pl.when(pl.program_id(0) == 0)
        def _():
            st_b[...] = st

        @pl.when(pl.program_id(0) != 0)
        def _():
            st_b[...] = st_b[...] + st

    return pl.pallas_call(
        body,
        grid=(nb,),
        in_specs=[
            pl.BlockSpec((be, 2 * hin), lambda i: (i, 0)),
            pl.BlockSpec((1, 2 * hin), lambda i: (0, 0)),
            pl.BlockSpec((1, 2 * hin), lambda i: (0, 0)),
            pl.BlockSpec((2 * hin, 2 * hout), lambda i: (0, 0)),
            pl.BlockSpec((1, 2 * hout), lambda i: (0, 0)),
        ],
        out_specs=[
            pl.BlockSpec((be, 2 * hout), lambda i: (i, 0)),
            pl.BlockSpec((2, 2 * hout), lambda i: (0, 0)),
        ],
        out_shape=[
            jax.ShapeDtypeStruct((e2, 2 * hout), jnp.float32),
            jax.ShapeDtypeStruct((2, 2 * hout), jnp.float32),
        ],
    )(z1p, s, t, w, b)


def _tc_mlp_out(z2p, s, t, w, b, e, hin, hout):
    e2 = e // 2
    be = 6400
    nb = e2 // be

    def body(z_b, s_r, t_r, w_r, b_r, ol_b, oh_b):
        zn = z_b[...] * s_r[...] + t_r[...]
        o = (jnp.dot(zn, w_r[...],
                     preferred_element_type=jnp.float32) + b_r[...])
        ot = o.T
        ol_b[...] = ot[:hout]
        oh_b[...] = ot[hout:]

    return pl.pallas_call(
        body,
        grid=(nb,),
        in_specs=[
            pl.BlockSpec((be, 2 * hin), lambda i: (i, 0)),
            pl.BlockSpec((1, 2 * hin), lambda i: (0, 0)),
            pl.BlockSpec((1, 2 * hin), lambda i: (0, 0)),
            pl.BlockSpec((2 * hin, 2 * hout), lambda i: (0, 0)),
            pl.BlockSpec((1, 2 * hout), lambda i: (0, 0)),
        ],
        out_specs=[
            pl.BlockSpec((hout, be), lambda i: (0, i)),
            pl.BlockSpec((hout, be), lambda i: (0, i)),
        ],
        out_shape=[
            jax.ShapeDtypeStruct((hout, e2), jnp.float32),
            jax.ShapeDtypeStruct((hout, e2), jnp.float32),
        ],
    )(z2p, s, t, w, b)


def kernel(x, edge_index, W1, b1, W2, b2, W3, b3, eW1, eb1, g1, be1, eW2,
           eb2, g2, be2, eW3, eb3):
    n, d = x.shape
    e = edge_index.shape[1]
    h = W1.shape[1]
    ew = e // NW
    ch = ew // C
    np_ = ((n + 2559) // 2560) * 2560

    src3 = edge_index[0].reshape(NW, ch, C)
    dst3 = edge_index[1].reshape(NW, ch, C)
    zeros64 = jnp.zeros((np_, h), jnp.float32)
    zeros16 = jnp.zeros((np_, 16), jnp.float32)
    ones = jnp.ones((C, 16), jnp.float32)
    xp = jnp.pad(x, ((0, np_ - n), (0, 0)))

    degp = _deg_partials(ones, dst3, zeros16, np_, ch)
    g_1, u_1, di = _tc_in(xp, W1, degp, np_, d, h)
    acc1 = _seg_sum(u_1, src3, dst3, zeros64, np_, h, ch)
    g_2, u_2 = _tc_conv(acc1, g_1, di, b1.reshape(1, h), W2, np_, h)
    acc2 = _seg_sum(u_2, src3, dst3, zeros64, np_, h, ch)
    g_3, u_3 = _tc_conv(acc2, g_2, di, b2.reshape(1, h), W3, np_, h)
    acc3 = _seg_sum(u_3, src3, dst3, zeros64, np_, h, ch)
    P, Q = _tc_pq(acc3, g_3, di, b3.reshape(1, h), eW1[:h], eW1[h:],
                  eb1.reshape(1, h), np_, h)

    e2 = e // 2
    c2 = C // 2
    srcm = jnp.concatenate([edge_index[0, :e2].reshape(NW, ch, c2),
                            edge_index[0, e2:].reshape(NW, ch, c2)], axis=2)
    dstm = jnp.concatenate([edge_index[1, :e2].reshape(NW, ch, c2),
                            edge_index[1, e2:].reshape(NW, ch, c2)], axis=2)
    z1, st1 = _edge_head(P, Q, srcm, dstm, np_, h, e, ch)

    ssum = st1[:, :4, :].reshape(NW, h).sum(0)
    ssq = st1[:, 4:, :].reshape(NW, h).sum(0)
    mean1 = ssum / e
    var1 = ssq / e - mean1 * mean1
    s1 = g1 / jnp.sqrt(var1 + 1e-5)
    t1 = be1 - mean1 * s1

    h2 = eW2.shape[1]
    zW2 = jnp.zeros_like(eW2)
    W2p = jnp.concatenate([jnp.concatenate([eW2, zW2], 1),
                           jnp.concatenate([zW2, eW2], 1)], 0)
    z2, st2p = _tc_mlp_mid(z1, jnp.tile(s1, 2).reshape(1, -1),
                           jnp.tile(t1, 2).reshape(1, -1), W2p,
                           jnp.tile(eb2, 2).reshape(1, -1), e, h, h2)

    st2 = st2p[:, :h2] + st2p[:, h2:]
    mean2 = st2[0] / e
    var2 = st2[1] / e - mean2 * mean2
    s2 = g2 / jnp.sqrt(var2 + 1e-5)
    t2 = be2 - mean2 * s2

    h3 = eW3.shape[1]
    zW3 = jnp.zeros_like(eW3)
    W3p = jnp.concatenate([jnp.concatenate([eW3, zW3], 1),
                           jnp.concatenate([zW3, eW3], 1)], 0)
    outTlo, outThi = _tc_mlp_out(z2, jnp.tile(s2, 2).reshape(1, -1),
                                 jnp.tile(t2, 2).reshape(1, -1), W3p,
                                 jnp.tile(eb3, 2).reshape(1, -1), e, h2, h3)
    outT = jnp.concatenate([outTlo, outThi], axis=1)
    return outT.T


# speedup vs baseline: 24.4176x; 1.2537x over previous
import functools

import jax
import jax.numpy as jnp
from jax import lax
from jax.experimental import pallas as pl
from jax.experimental.pallas import tpu as pltpu
from jax.experimental.pallas import tpu_sc as plsc

NC = 2
NS = 16
NW = NC * NS

C = 80
NB = 5


def _mesh():
    return plsc.VectorSubcoreMesh(core_axis_name="c", subcore_axis_name="s")


def _deg_partials(ones, dst3, zeros16, n, ch):
    rows = n // NS

    @functools.partial(
        pl.kernel,
        out_type=jax.ShapeDtypeStruct((NC, n, 16), jnp.float32),
        mesh=_mesh(),
        compiler_params=pltpu.CompilerParams(use_tc_tiling_on_sc=False),
        scratch_types=[
            pltpu.VMEM((ch, C), jnp.int32),
            pltpu.VMEM((C, 16), jnp.float32),
            pltpu.VMEM_SHARED((n, 16), jnp.float32),
            pltpu.SemaphoreType.DMA,
        ],
    )
    def k(ones_hbm, dst_hbm, zero_hbm, out_hbm, dst_v, ones_v, deg_sh, sem):
        c = lax.axis_index("c")
        s = lax.axis_index("s")
        wid = s * NC + c
        pltpu.sync_copy(zero_hbm.at[pl.ds(s * rows, rows)],
                        deg_sh.at[pl.ds(s * rows, rows)])
        pltpu.sync_copy(dst_hbm.at[wid], dst_v)
        pltpu.sync_copy(ones_hbm, ones_v)
        plsc.subcore_barrier()

        def body(j, carry):
            pltpu.async_copy(ones_v, deg_sh.at[dst_v.at[j]], sem, add=True)
            return carry

        lax.fori_loop(0, ch, body, 0)

        def drain(j, carry):
            pltpu.make_async_copy(ones_v, deg_sh.at[dst_v.at[0]], sem).wait()
            return carry

        lax.fori_loop(0, ch, drain, 0)
        plsc.subcore_barrier()
        pltpu.sync_copy(deg_sh.at[pl.ds(s * rows, rows)],
                        out_hbm.at[c, pl.ds(s * rows, rows)])

    return k(ones, dst3, zeros16)


def _seg_sum(u, src3, dst3, zeros64, n, h, ch):
    rows = n // NS

    @functools.partial(
        pl.kernel,
        out_type=jax.ShapeDtypeStruct((NC, n, h), jnp.float32),
        mesh=_mesh(),
        compiler_params=pltpu.CompilerParams(use_tc_tiling_on_sc=False),
        scratch_types=[
            pltpu.VMEM((ch, C), jnp.int32),
            pltpu.VMEM((ch, C), jnp.int32),
            pltpu.VMEM((NB, C, h), jnp.float32),
            pltpu.VMEM_SHARED((n, h), jnp.float32),
            pltpu.SemaphoreType.DMA((NB,)),
            pltpu.SemaphoreType.DMA((NB,)),
        ],
    )
    def k(u_hbm, src_hbm, dst_hbm, zero_hbm, out_hbm, src_v, dst_v, gath_v,
          acc_sh, sem, ssem):
        c = lax.axis_index("c")
        s = lax.axis_index("s")
        wid = s * NC + c
        pltpu.sync_copy(zero_hbm.at[pl.ds(s * rows, rows)],
                        acc_sh.at[pl.ds(s * rows, rows)])
        pltpu.sync_copy(src_hbm.at[wid], src_v)
        pltpu.sync_copy(dst_hbm.at[wid], dst_v)
        plsc.subcore_barrier()

        LA = 2
        for b in range(LA):
            pltpu.async_copy(u_hbm.at[src_v.at[b]], gath_v.at[b], sem.at[b])

        def body(i, carry):
            j = i * NB
            for b in range(NB):
                jj = j + b
                bn = (b + LA) % NB
                nxt = jj + LA

                @pl.when(nxt < ch)
                def _():
                    @pl.when(nxt >= NB)
                    def _():
                        pltpu.make_async_copy(
                            gath_v.at[bn], acc_sh.at[dst_v.at[0]],
                            ssem.at[bn]).wait()

                    pltpu.async_copy(u_hbm.at[src_v.at[nxt]], gath_v.at[bn],
                                     sem.at[bn])

                pltpu.make_async_copy(u_hbm.at[src_v.at[jj]], gath_v.at[b],
                                      sem.at[b]).wait()
                pltpu.async_copy(gath_v.at[b], acc_sh.at[dst_v.at[jj]],
                                 ssem.at[b], add=True)
            return carry

        lax.fori_loop(0, ch // NB, body, 0)
        for b in range(NB):
            pltpu.make_async_copy(gath_v.at[b], acc_sh.at[dst_v.at[0]],
                                  ssem.at[b]).wait()
        plsc.subcore_barrier()
        pltpu.sync_copy(acc_sh.at[pl.ds(s * rows, rows)],
                        out_hbm.at[c, pl.ds(s * rows, rows)])

    return k(u, src3, dst3, zeros64)


def _edge_head(p, q, srcm, dstm, n, h, e, ch):
    e2 = e // 2
    c2 = C // 2
    ew2 = e2 // NW

    @functools.partial(
        pl.kernel,
        out_type=(
            jax.ShapeDtypeStruct((e2, 2 * h), jnp.float32),
            jax.ShapeDtypeStruct((NW, 8, 16), jnp.float32),
        ),
        mesh=_mesh(),
        compiler_params=pltpu.CompilerParams(use_tc_tiling_on_sc=False),
        scratch_types=[
            pltpu.VMEM((ch, C), jnp.int32),
            pltpu.VMEM((ch, C), jnp.int32),
            pltpu.VMEM((NB, C, h), jnp.float32),
            pltpu.VMEM((NB, C, h), jnp.float32),
            pltpu.VMEM((NB, c2, 2 * h), jnp.float32),
            pltpu.VMEM((8, 16), jnp.float32),
            pltpu.SemaphoreType.DMA((NB,)),
            pltpu.SemaphoreType.DMA((NB,)),
            pltpu.SemaphoreType.DMA((NB,)),
        ],
    )
    def k(p_hbm, q_hbm, src_hbm, dst_hbm, z1_hbm, st_hbm, src_v, dst_v, a_v,
          b_v, z_v, st_v, sga, sgb, sw):
        c = lax.axis_index("c")
        s = lax.axis_index("s")
        wid = s * NC + c
        pltpu.sync_copy(src_hbm.at[wid], src_v)
        pltpu.sync_copy(dst_hbm.at[wid], dst_v)
        zero = jnp.zeros((16,), jnp.float32)
        LA = 2

        for b in range(LA):
            pltpu.async_copy(p_hbm.at[src_v.at[b]], a_v.at[b], sga.at[b])
            pltpu.async_copy(q_hbm.at[dst_v.at[b]], b_v.at[b], sgb.at[b])

        def outer(i, carry):
            j = i * NB
            for b in range(NB):
                jj = j + b
                bn = (b + LA) % NB
                nxt = jj + LA

                @pl.when(nxt < ch)
                def _():
                    @pl.when(nxt >= NB)
                    def _():
                        pltpu.make_async_copy(
                            z_v.at[bn], z1_hbm.at[pl.ds(0, c2)],
                            sw.at[bn]).wait()

                    pltpu.async_copy(p_hbm.at[src_v.at[nxt]], a_v.at[bn],
                                     sga.at[bn])
                    pltpu.async_copy(q_hbm.at[dst_v.at[nxt]], b_v.at[bn],
                                     sgb.at[bn])

                pltpu.make_async_copy(p_hbm.at[src_v.at[jj]], a_v.at[b],
                                      sga.at[b]).wait()
                pltpu.make_async_copy(q_hbm.at[dst_v.at[jj]], b_v.at[b],
                                      sgb.at[b]).wait()

                def row(pr, cr):
                    acc = list(cr)
                    for half in range(2):
                        r = 2 * pr + half
                        al = [a_v[b, r, pl.ds(kk * 16, 16)] for kk in range(4)]
                        bl = [b_v[b, r, pl.ds(kk * 16, 16)] for kk in range(4)]
                        ah = [a_v[b, c2 + r, pl.ds(kk * 16, 16)]
                              for kk in range(4)]
                        bh = [b_v[b, c2 + r, pl.ds(kk * 16, 16)]
                              for kk in range(4)]
                        zl = [jnp.maximum(al[kk] + bl[kk], 0.0)
                              for kk in range(4)]
                        zh = [jnp.maximum(ah[kk] + bh[kk], 0.0)
                              for kk in range(4)]
                        for kk in range(4):
                            z_v[b, r, pl.ds(kk * 16, 16)] = zl[kk]
                            z_v[b, r, pl.ds(h + kk * 16, 16)] = zh[kk]
                            acc[kk] = acc[kk] + zl[kk]
                            acc[4 + kk] = acc[4 + kk] + zl[kk] * zl[kk]
                            acc[8 + kk] = acc[8 + kk] + zh[kk]
                            acc[12 + kk] = acc[12 + kk] + zh[kk] * zh[kk]
                    return tuple(acc)

                carry = lax.fori_loop(0, c2 // 2, row, carry)
                pltpu.async_copy(z_v.at[b],
                                 z1_hbm.at[pl.ds(wid * ew2 + jj * c2, c2)],
                                 sw.at[b])
            return carry

        carry = lax.fori_loop(0, ch // NB, outer, (zero,) * 16)
        for b in range(NB):
            pltpu.make_async_copy(z_v.at[b], z1_hbm.at[pl.ds(0, c2)],
                                  sw.at[b]).wait()
        for kk in range(4):
            st_v[kk, :] = carry[kk] + carry[8 + kk]
            st_v[4 + kk, :] = carry[4 + kk] + carry[12 + kk]
        pltpu.sync_copy(st_v, st_hbm.at[wid])

    return k(p, q, srcm, dstm)


def _tc_in(x, w1, degp, n, d, h):
    bn = 2560

    def body(x_b, w_r, deg_b, g_b, u_b, di_b):
        deg = deg_b[...]
        dtot = deg[0, :, 0:1] + deg[1, :, 0:1] + 1.0
        di = lax.rsqrt(dtot)
        g = jnp.dot(x_b[...], w_r[...], preferred_element_type=jnp.float32)
        g_b[...] = g
        u_b[...] = g * di
        di_b[...] = jnp.broadcast_to(di, di_b.shape)

    return pl.pallas_call(
        body,
        grid=(n // bn,),
        in_specs=[
            pl.BlockSpec((bn, d), lambda i: (i, 0)),
            pl.BlockSpec((d, h), lambda i: (0, 0)),
            pl.BlockSpec((NC, bn, 16), lambda i: (0, i, 0)),
        ],
        out_specs=[
            pl.BlockSpec((bn, h), lambda i: (i, 0)),
            pl.BlockSpec((bn, h), lambda i: (i, 0)),
            pl.BlockSpec((bn, 16), lambda i: (i, 0)),
        ],
        out_shape=[
            jax.ShapeDtypeStruct((n, h), jnp.float32),
            jax.ShapeDtypeStruct((n, h), jnp.float32),
            jax.ShapeDtypeStruct((n, 16), jnp.float32),
        ],
    )(x, w1, degp)


def _tc_conv(acc, g, di, b, w, n, h):
    bn = 2560

    def body(acc_b, g_b, di_b, b_r, w_r, gn_b, un_b):
        a = acc_b[...]
        dv = di_b[...][:, 0:1]
        hh = jnp.maximum(dv * (a[0] + a[1]) + dv * dv * g_b[...] + b_r[...],
                         0.0)
        gn = jnp.dot(hh, w_r[...], preferred_element_type=jnp.float32)
        gn_b[...] = gn
        un_b[...] = gn * dv

    return pl.pallas_call(
        body,
        grid=(n // bn,),
        in_specs=[
            pl.BlockSpec((NC, bn, h), lambda i: (0, i, 0)),
            pl.BlockSpec((bn, h), lambda i: (i, 0)),
            pl.BlockSpec((bn, 16), lambda i: (i, 0)),
            pl.BlockSpec((1, h), lambda i: (0, 0)),
            pl.BlockSpec((h, h), lambda i: (0, 0)),
        ],
        out_specs=[
            pl.BlockSpec((bn, h), lambda i: (i, 0)),
            pl.BlockSpec((bn, h), lambda i: (i, 0)),
        ],
        out_shape=[
            jax.ShapeDtypeStruct((n, h), jnp.float32),
            jax.ShapeDtypeStruct((n, h), jnp.float32),
        ],
    )(acc, g, di, b, w)


def _tc_pq(acc, g, di, b, ea, eb, eb1, n, h):
    bn = 2560

    def body(acc_b, g_b, di_b, b_r, ea_r, eb_r, eb1_r, p_b, q_b):
        a = acc_b[...]
        dv = di_b[...][:, 0:1]
        h3 = dv * (a[0] + a[1]) + dv * dv * g_b[...] + b_r[...]
        p_b[...] = (jnp.dot(h3, ea_r[...], preferred_element_type=jnp.float32)
                    + eb1_r[...])
        q_b[...] = jnp.dot(h3, eb_r[...], preferred_element_type=jnp.float32)

    return pl.pallas_call(
        body,
        grid=(n // bn,),
        in_specs=[
            pl.BlockSpec((NC, bn, h), lambda i: (0, i, 0)),
            pl.BlockSpec((bn, h), lambda i: (i, 0)),
            pl.BlockSpec((bn, 16), lambda i: (i, 0)),
            pl.BlockSpec((1, h), lambda i: (0, 0)),
            pl.BlockSpec((h, h), lambda i: (0, 0)),
            pl.BlockSpec((h, h), lambda i: (0, 0)),
            pl.BlockSpec((1, h), lambda i: (0, 0)),
        ],
        out_specs=[
            pl.BlockSpec((bn, h), lambda i: (i, 0)),
            pl.BlockSpec((bn, h), lambda i: (i, 0)),
        ],
        out_shape=[
            jax.ShapeDtypeStruct((n, h), jnp.float32),
            jax.ShapeDtypeStruct((n, h), jnp.float32),
        ],
    )(acc, g, di, b, ea, eb, eb1)


def _tc_mlp_mid(z1p, s, t, w, b, e, hin, hout):
    e2 = e // 2
    be = 6400
    nb = e2 // be

    def body(z_b, s_r, t_r, w_r, b_r, z2_b, st_b):
        zn = z_b[...] * s_r[...] + t_r[...]
        z2 = jnp.maximum(
            jnp.dot(zn, w_r[...], preferred_element_type=jnp.float32)
            + b_r[...], 0.0)
        z2_b[...] = z2
        st = jnp.concatenate(
            [jnp.sum(z2, axis=0, keepdims=True),
             jnp.sum(z2 * z2, axis=0, keepdims=True)], axis=0)

        @pl.when(pl.program_id(0) == 0)
        def _():
            st_b[...] = st

        @pl.when(pl.program_id(0) != 0)
        def _():
            st_b[...] = st_b[...] + st

    return pl.pallas_call(
        body,
        grid=(nb,),
        in_specs=[
            pl.BlockSpec((be, 2 * hin), lambda i: (i, 0)),
            pl.BlockSpec((1, 2 * hin), lambda i: (0, 0)),
            pl.BlockSpec((1, 2 * hin), lambda i: (0, 0)),
            pl.BlockSpec((2 * hin, 2 * hout), lambda i: (0, 0)),
            pl.BlockSpec((1, 2 * hout), lambda i: (0, 0)),
        ],
        out_specs=[
            pl.BlockSpec((be, 2 * hout), lambda i: (i, 0)),
            pl.BlockSpec((2, 2 * hout), lambda i: (0, 0)),
        ],
        out_shape=[
            jax.ShapeDtypeStruct((e2, 2 * hout), jnp.float32),
            jax.ShapeDtypeStruct((2, 2 * hout), jnp.float32),
        ],
    )(z1p, s, t, w, b)


def _tc_mlp_out(z2p, s, t, w, b, e, hin, hout):
    e2 = e // 2
    be = 6400
    nb = e2 // be

    def body(z_b, s_r, t_r, w_r, b_r, ol_b, oh_b):
        zn = z_b[...] * s_r[...] + t_r[...]
        o = (jnp.dot(zn, w_r[...],
                     preferred_element_type=jnp.float32) + b_r[...])
        ot = o.T
        ol_b[...] = ot[:hout]
        oh_b[...] = ot[hout:]

    return pl.pallas_call(
        body,
        grid=(nb,),
        in_specs=[
            pl.BlockSpec((be, 2 * hin), lambda i: (i, 0)),
            pl.BlockSpec((1, 2 * hin), lambda i: (0, 0)),
            pl.BlockSpec((1, 2 * hin), lambda i: (0, 0)),
            pl.BlockSpec((2 * hin, 2 * hout), lambda i: (0, 0)),
            pl.BlockSpec((1, 2 * hout), lambda i: (0, 0)),
        ],
        out_specs=[
            pl.BlockSpec((hout, be), lambda i: (0, i)),
            pl.BlockSpec((hout, be), lambda i: (0, i)),
        ],
        out_shape=[
            jax.ShapeDtypeStruct((hout, e2), jnp.float32),
            jax.ShapeDtypeStruct((hout, e2), jnp.float32),
        ],
    )(z2p, s, t, w, b)


def kernel(x, edge_index, W1, b1, W2, b2, W3, b3, eW1, eb1, g1, be1, eW2,
           eb2, g2, be2, eW3, eb3):
    n, d = x.shape
    e = edge_index.shape[1]
    h = W1.shape[1]
    ew = e // NW
    ch = ew // C
    np_ = ((n + 2559) // 2560) * 2560

    src3 = edge_index[0].reshape(NW, ch, C)
    dst3 = edge_index[1].reshape(NW, ch, C)
    zeros64 = jnp.zeros((np_, h), jnp.float32)
    zeros16 = jnp.zeros((np_, 16), jnp.float32)
    ones = jnp.ones((C, 16), jnp.float32)
    xp = jnp.pad(x, ((0, np_ - n), (0, 0)))

    degp = _deg_partials(ones, dst3, zeros16, np_, ch)
    g_1, u_1, di = _tc_in(xp, W1, degp, np_, d, h)
    acc1 = _seg_sum(u_1, src3, dst3, zeros64, np_, h, ch)
    g_2, u_2 = _tc_conv(acc1, g_1, di, b1.reshape(1, h), W2, np_, h)
    acc2 = _seg_sum(u_2, src3, dst3, zeros64, np_, h, ch)
    g_3, u_3 = _tc_conv(acc2, g_2, di, b2.reshape(1, h), W3, np_, h)
    acc3 = _seg_sum(u_3, src3, dst3, zeros64, np_, h, ch)
    P, Q = _tc_pq(acc3, g_3, di, b3.reshape(1, h), eW1[:h], eW1[h:],
                  eb1.reshape(1, h), np_, h)

    e2 = e // 2
    c2 = C // 2
    srcm = jnp.concatenate([edge_index[0, :e2].reshape(NW, ch, c2),
                            edge_index[0, e2:].reshape(NW, ch, c2)], axis=2)
    dstm = jnp.concatenate([edge_index[1, :e2].reshape(NW, ch, c2),
                            edge_index[1, e2:].reshape(NW, ch, c2)], axis=2)
    z1, st1 = _edge_head(P, Q, srcm, dstm, np_, h, e, ch)

    ssum = st1[:, :4, :].reshape(NW, h).sum(0)
    ssq = st1[:, 4:, :].reshape(NW, h).sum(0)
    mean1 = ssum / e
    var1 = ssq / e - mean1 * mean1
    s1 = g1 / jnp.sqrt(var1 + 1e-5)
    t1 = be1 - mean1 * s1

    h2 = eW2.shape[1]
    zW2 = jnp.zeros_like(eW2)
    W2p = jnp.concatenate([jnp.concatenate([eW2, zW2], 1),
                           jnp.concatenate([zW2, eW2], 1)], 0)
    z2, st2p = _tc_mlp_mid(z1, jnp.tile(s1, 2).reshape(1, -1),
                           jnp.tile(t1, 2).reshape(1, -1), W2p,
                           jnp.tile(eb2, 2).reshape(1, -1), e, h, h2)

    st2 = st2p[:, :h2] + st2p[:, h2:]
    mean2 = st2[0] / e
    var2 = st2[1] / e - mean2 * mean2
    s2 = g2 / jnp.sqrt(var2 + 1e-5)
    t2 = be2 - mean2 * s2

    h3 = eW3.shape[1]
    zW3 = jnp.zeros_like(eW3)
    W3p = jnp.concatenate([jnp.concatenate([eW3, zW3], 1),
                           jnp.concatenate([zW3, eW3], 1)], 0)
    outTlo, outThi = _tc_mlp_out(z2, jnp.tile(s2, 2).reshape(1, -1),
                                 jnp.tile(t2, 2).reshape(1, -1), W3p,
                                 jnp.tile(eb3, 2).reshape(1, -1), e, h2, h3)
    outT = jnp.concatenate([outTlo, outThi], axis=1)
    return outT.T
